# Initial kernel scaffold; baseline (speedup 1.0000x reference)
import jax
import jax.numpy as jnp
from jax.experimental import pallas as pl


def kernel(x, edge_index, graph_id, batch, W1, b1, W2, b2, Wfc, bfc):
    raise NotImplementedError("write your pallas kernel here")


# speedup vs baseline: 11.1815x; 11.1815x over previous
import functools

import jax
import jax.numpy as jnp
from jax import lax
from jax.experimental import pallas as pl
from jax.experimental.pallas import tpu as pltpu
from jax.experimental.pallas import tpu_sc as plsc

N = 50000
E = 800000
H = 64
B = 512
NCLS = 7
NC = 2
NS = 16
NPAD = 50176
RPT = NPAD // NS
NBLK = NPAD // 128
CHUNK = 128
PW = 80


def _sc_degree(dst, ones_rows, z16):
    ept = E // (NC * NS)
    nfull = ept // CHUNK
    tail = ept - nfull * CHUNK
    mesh = plsc.VectorSubcoreMesh(core_axis_name="c", subcore_axis_name="s")

    @functools.partial(
        pl.kernel,
        out_type=jax.ShapeDtypeStruct((NC, NPAD, 16), jnp.float32),
        mesh=mesh,
        compiler_params=pltpu.CompilerParams(use_tc_tiling_on_sc=False),
        scratch_types=[
            pltpu.VMEM((CHUNK,), jnp.int32),
            pltpu.VMEM((tail,), jnp.int32),
            pltpu.VMEM((CHUNK, 16), jnp.float32),
            pltpu.VMEM((tail, 16), jnp.float32),
            pltpu.VMEM_SHARED((NPAD, 16), jnp.float32),
        ],
    )
    def k(dst_h, ones_h, z_h, out_h, idx_v, idxt_v, rows_v, rowst_v, acc):
        c = lax.axis_index("c")
        s = lax.axis_index("s")
        pltpu.sync_copy(z_h.at[pl.ds(s * RPT, RPT)], acc.at[pl.ds(s * RPT, RPT)])
        pltpu.sync_copy(ones_h, rows_v)
        pltpu.sync_copy(ones_h.at[pl.ds(0, tail)], rowst_v)
        plsc.subcore_barrier()
        base = c * (E // NC) + s * ept

        def body(g, carry):
            off = base + g * CHUNK
            pltpu.sync_copy(dst_h.at[pl.ds(off, CHUNK)], idx_v)
            pltpu.sync_copy(rows_v, acc.at[idx_v], add=True)
            return carry

        lax.fori_loop(0, nfull, body, 0)
        off = base + nfull * CHUNK
        pltpu.sync_copy(dst_h.at[pl.ds(off, tail)], idxt_v)
        pltpu.sync_copy(rowst_v, acc.at[idxt_v], add=True)
        plsc.subcore_barrier()
        pltpu.sync_copy(acc.at[pl.ds(s * RPT, RPT)],
                        out_h.at[c, pl.ds(s * RPT, RPT)])

    return k(dst, ones_rows, z16)


def _sc_agg16(src, dst, table, z16):
    ept = E // (NC * NS)
    nfull = ept // CHUNK
    tail = ept - nfull * CHUNK
    mesh = plsc.VectorSubcoreMesh(core_axis_name="c", subcore_axis_name="s")

    @functools.partial(
        pl.kernel,
        out_type=jax.ShapeDtypeStruct((NC, NPAD, 16), jnp.float32),
        mesh=mesh,
        compiler_params=pltpu.CompilerParams(use_tc_tiling_on_sc=False),
        scratch_types=[
            pltpu.VMEM((CHUNK,), jnp.int32),
            pltpu.VMEM((CHUNK,), jnp.int32),
            pltpu.VMEM((tail,), jnp.int32),
            pltpu.VMEM((tail,), jnp.int32),
            pltpu.VMEM((CHUNK, 16), jnp.float32),
            pltpu.VMEM((tail, 16), jnp.float32),
            pltpu.VMEM_SHARED((NPAD, 16), jnp.float32),
        ],
    )
    def k(src_h, dst_h, tab_h, z_h, out_h,
          isv, idv, istv, idtv, rows_v, rowst_v, acc):
        c = lax.axis_index("c")
        s = lax.axis_index("s")
        pltpu.sync_copy(z_h.at[pl.ds(s * RPT, RPT)], acc.at[pl.ds(s * RPT, RPT)])
        plsc.subcore_barrier()
        base = c * (E // NC) + s * ept

        def body(g, carry):
            off = base + g * CHUNK
            pltpu.sync_copy(src_h.at[pl.ds(off, CHUNK)], isv)
            pltpu.sync_copy(dst_h.at[pl.ds(off, CHUNK)], idv)
            pltpu.sync_copy(tab_h.at[isv], rows_v)
            pltpu.sync_copy(rows_v, acc.at[idv], add=True)
            return carry

        lax.fori_loop(0, nfull, body, 0)
        off = base + nfull * CHUNK
        pltpu.sync_copy(src_h.at[pl.ds(off, tail)], istv)
        pltpu.sync_copy(dst_h.at[pl.ds(off, tail)], idtv)
        pltpu.sync_copy(tab_h.at[istv], rowst_v)
        pltpu.sync_copy(rowst_v, acc.at[idtv], add=True)
        plsc.subcore_barrier()
        pltpu.sync_copy(acc.at[pl.ds(s * RPT, RPT)],
                        out_h.at[c, pl.ds(s * RPT, RPT)])

    return k(src, dst, table, z16)


def _sc_agg32(srcs2, dst, y1cat, z32):
    ept = E // NS
    nfull = ept // CHUNK
    tail = ept - nfull * CHUNK
    mesh = plsc.VectorSubcoreMesh(core_axis_name="c", subcore_axis_name="s")

    @functools.partial(
        pl.kernel,
        out_type=jax.ShapeDtypeStruct((NC, NPAD, 32), jnp.float32),
        mesh=mesh,
        compiler_params=pltpu.CompilerParams(use_tc_tiling_on_sc=False),
        scratch_types=[
            pltpu.VMEM((CHUNK,), jnp.int32),
            pltpu.VMEM((CHUNK,), jnp.int32),
            pltpu.VMEM((tail,), jnp.int32),
            pltpu.VMEM((tail,), jnp.int32),
            pltpu.VMEM((CHUNK, 32), jnp.float32),
            pltpu.VMEM((tail, 32), jnp.float32),
            pltpu.VMEM_SHARED((NPAD, 32), jnp.float32),
        ],
    )
    def k(srcs_h, dst_h, tab_h, z_h, out_h,
          isv, idv, istv, idtv, rows_v, rowst_v, acc):
        c = lax.axis_index("c")
        s = lax.axis_index("s")
        pltpu.sync_copy(z_h.at[pl.ds(s * RPT, RPT)], acc.at[pl.ds(s * RPT, RPT)])
        plsc.subcore_barrier()
        base = s * ept

        def body(g, carry):
            off = base + g * CHUNK
            pltpu.sync_copy(srcs_h.at[c, pl.ds(off, CHUNK)], isv)
            pltpu.sync_copy(dst_h.at[pl.ds(off, CHUNK)], idv)
            pltpu.sync_copy(tab_h.at[isv], rows_v)
            pltpu.sync_copy(rows_v, acc.at[idv], add=True)
            return carry

        lax.fori_loop(0, nfull, body, 0)
        off = base + nfull * CHUNK
        pltpu.sync_copy(srcs_h.at[c, pl.ds(off, tail)], istv)
        pltpu.sync_copy(dst_h.at[pl.ds(off, tail)], idtv)
        pltpu.sync_copy(tab_h.at[istv], rowst_v)
        pltpu.sync_copy(rowst_v, acc.at[idtv], add=True)
        plsc.subcore_barrier()
        pltpu.sync_copy(acc.at[pl.ds(s * RPT, RPT)],
                        out_h.at[c, pl.ds(s * RPT, RPT)])

    return k(srcs2, dst, y1cat, z32)


def _sc_pool(h2pad, seg, z80):
    rpc = NPAD // NC
    rpt = rpc // NS
    nfull = rpt // CHUNK
    tail = rpt - nfull * CHUNK
    arows = (2 * B) // NS
    mesh = plsc.VectorSubcoreMesh(core_axis_name="c", subcore_axis_name="s")

    @functools.partial(
        pl.kernel,
        out_type=jax.ShapeDtypeStruct((NC, 2 * B, PW), jnp.float32),
        mesh=mesh,
        compiler_params=pltpu.CompilerParams(use_tc_tiling_on_sc=False),
        scratch_types=[
            pltpu.VMEM((CHUNK,), jnp.int32),
            pltpu.VMEM((tail,), jnp.int32),
            pltpu.VMEM((CHUNK, PW), jnp.float32),
            pltpu.VMEM((tail, PW), jnp.float32),
            pltpu.VMEM_SHARED((2 * B, PW), jnp.float32),
        ],
    )
    def k(h_h, seg_h, z_h, out_h, idx_v, idxt_v, rows_v, rowst_v, acc):
        c = lax.axis_index("c")
        s = lax.axis_index("s")
        pltpu.sync_copy(z_h.at[pl.ds(s * arows, arows)],
                        acc.at[pl.ds(s * arows, arows)])
        plsc.subcore_barrier()
        base = c * rpc + s * rpt

        def body(g, carry):
            off = base + g * CHUNK
            pltpu.sync_copy(seg_h.at[pl.ds(off, CHUNK)], idx_v)
            pltpu.sync_copy(h_h.at[pl.ds(off, CHUNK)], rows_v)
            pltpu.sync_copy(rows_v, acc.at[idx_v], add=True)
            return carry

        lax.fori_loop(0, nfull, body, 0)
        off = base + nfull * CHUNK
        pltpu.sync_copy(seg_h.at[pl.ds(off, tail)], idxt_v)
        pltpu.sync_copy(h_h.at[pl.ds(off, tail)], rowst_v)
        pltpu.sync_copy(rowst_v, acc.at[idxt_v], add=True)
        plsc.subcore_barrier()
        pltpu.sync_copy(acc.at[pl.ds(s * arows, arows)],
                        out_h.at[c, pl.ds(s * arows, arows)])

    return k(h2pad, seg, z80)


def _tc_prep(degp, x_pad, batch2d, gid2d):
    def body(degp_ref, x_ref, b_ref, g_ref, dinv_ref, y0_ref, seg_ref):
        deg16 = degp_ref[0] + degp_ref[1] + 1.0
        dinv16 = lax.rsqrt(deg16)
        dinv_ref[...] = dinv16
        y0 = x_ref[...] * dinv16[:, 0:2]
        y0_ref[...] = jnp.concatenate(
            [y0, jnp.zeros((128, 14), jnp.float32)], axis=1)
        seg_ref[...] = b_ref[...] + B * g_ref[...]

    return pl.pallas_call(
        body,
        grid=(NBLK,),
        in_specs=[
            pl.BlockSpec((NC, 128, 16), lambda i: (0, i, 0)),
            pl.BlockSpec((128, 2), lambda i: (i, 0)),
            pl.BlockSpec((1, 1, 128), lambda i: (i, 0, 0)),
            pl.BlockSpec((1, 1, 128), lambda i: (i, 0, 0)),
        ],
        out_specs=[
            pl.BlockSpec((128, 16), lambda i: (i, 0)),
            pl.BlockSpec((128, 16), lambda i: (i, 0)),
            pl.BlockSpec((1, 1, 128), lambda i: (i, 0, 0)),
        ],
        out_shape=[
            jax.ShapeDtypeStruct((NPAD, 16), jnp.float32),
            jax.ShapeDtypeStruct((NPAD, 16), jnp.float32),
            jax.ShapeDtypeStruct((NBLK, 1, 128), jnp.int32),
        ],
    )(degp, x_pad, batch2d, gid2d)


def _tc_layer1(s1p, y0pad, dinv16, W1, b1):
    def body(s1_ref, y0_ref, dv_ref, w1_ref, b1_ref, y1_ref):
        dinv16 = dv_ref[...]
        a1 = dinv16 * (s1_ref[0] + s1_ref[1] + y0_ref[...])
        h1 = a1[:, 0:1] * w1_ref[0:1, :] + a1[:, 1:2] * w1_ref[1:2, :]
        h1 = jnp.maximum(h1 + b1_ref[...], 0.0)
        y1 = dinv16[:, 0:1] * h1
        y1_ref[0] = y1[:, 0:32]
        y1_ref[1] = y1[:, 32:64]

    return pl.pallas_call(
        body,
        grid=(NBLK,),
        in_specs=[
            pl.BlockSpec((NC, 128, 16), lambda i: (0, i, 0)),
            pl.BlockSpec((128, 16), lambda i: (i, 0)),
            pl.BlockSpec((128, 16), lambda i: (i, 0)),
            pl.BlockSpec((2, H), lambda i: (0, 0)),
            pl.BlockSpec((1, H), lambda i: (0, 0)),
        ],
        out_specs=pl.BlockSpec((NC, 128, 32), lambda i: (0, i, 0)),
        out_shape=jax.ShapeDtypeStruct((NC, NPAD, 32), jnp.float32),
    )(s1p, y0pad, dinv16, W1, b1)


def _tc_layer2(s2p, y1p, dinv16, W2, b2):
    def body(s2_ref, y1_ref, dv_ref, w2_ref, b2_ref, out_ref):
        i = pl.program_id(0)
        s2 = jnp.concatenate([s2_ref[0] + y1_ref[0], s2_ref[1] + y1_ref[1]],
                             axis=1)
        a2 = dv_ref[...][:, 0:1] * s2
        h2 = jnp.dot(a2, w2_ref[...], preferred_element_type=jnp.float32)
        h2 = jnp.maximum(h2 + b2_ref[...], 0.0)
        row = i * 128 + lax.broadcasted_iota(jnp.int32, (128, 1), 0)
        valid = (row < N).astype(jnp.float32)
        out_ref[...] = jnp.concatenate(
            [h2 * valid, valid, jnp.zeros((128, PW - H - 1), jnp.float32)],
            axis=1)

    return pl.pallas_call(
        body,
        grid=(NBLK,),
        in_specs=[
            pl.BlockSpec((NC, 128, 32), lambda i: (0, i, 0)),
            pl.BlockSpec((NC, 128, 32), lambda i: (0, i, 0)),
            pl.BlockSpec((128, 16), lambda i: (i, 0)),
            pl.BlockSpec((H, H), lambda i: (0, 0)),
            pl.BlockSpec((1, H), lambda i: (0, 0)),
        ],
        out_specs=pl.BlockSpec((128, PW), lambda i: (i, 0)),
        out_shape=jax.ShapeDtypeStruct((NPAD, PW), jnp.float32),
    )(s2p, y1p, dinv16, W2, b2)


def _tc_final(poolp, Wfc, bfc):
    def body(p_ref, w_ref, b_ref, out_ref):
        P = p_ref[0] + p_ref[1]
        sums = P[:, 0:H]
        cnt = P[:, H:H + 1]
        p = sums / jnp.maximum(cnt, 1.0)
        combined = jnp.concatenate([p[0:B], p[B:2 * B]], axis=1)
        out_ref[...] = jnp.dot(combined, w_ref[...],
                               preferred_element_type=jnp.float32) + b_ref[...]

    return pl.pallas_call(
        body,
        grid=(1,),
        in_specs=[
            pl.BlockSpec((NC, 2 * B, PW), lambda i: (0, 0, 0)),
            pl.BlockSpec((2 * H, NCLS), lambda i: (0, 0)),
            pl.BlockSpec((1, NCLS), lambda i: (0, 0)),
        ],
        out_specs=pl.BlockSpec((B, NCLS), lambda i: (0, 0)),
        out_shape=jax.ShapeDtypeStruct((B, NCLS), jnp.float32),
    )(poolp, Wfc, bfc)


def kernel(x, edge_index, graph_id, batch, W1, b1, W2, b2, Wfc, bfc):
    src = edge_index[0].astype(jnp.int32)
    dst = edge_index[1].astype(jnp.int32)
    x_pad = jnp.pad(x, ((0, NPAD - N), (0, 0)))
    batch2d = jnp.pad(batch, (0, NPAD - N)).reshape(NBLK, 1, 128)
    gid2d = jnp.pad(graph_id, (0, NPAD - N)).reshape(NBLK, 1, 128)
    ones_rows = jnp.ones((CHUNK, 16), jnp.float32)
    z16 = jnp.zeros((NPAD, 16), jnp.float32)
    z32 = jnp.zeros((NPAD, 32), jnp.float32)
    z80 = jnp.zeros((2 * B, PW), jnp.float32)

    degp = _sc_degree(dst, ones_rows, z16)
    dinv16, y0pad, seg2d = _tc_prep(degp, x_pad, batch2d, gid2d)
    s1p = _sc_agg16(src, dst, y0pad, z16)
    y1p = _tc_layer1(s1p, y0pad, dinv16, W1, b1.reshape(1, H))
    srcs2 = jnp.stack([src, src + NPAD])
    y1cat = y1p.reshape(NC * NPAD, 32)
    s2p = _sc_agg32(srcs2, dst, y1cat, z32)
    h2pad = _tc_layer2(s2p, y1p, dinv16, W2, b2.reshape(1, H))
    poolp = _sc_pool(h2pad, seg2d.reshape(NPAD), z80)
    return _tc_final(poolp, Wfc, bfc.reshape(1, NCLS))


# speedup vs baseline: 12.6511x; 1.1314x over previous
import functools

import jax
import jax.numpy as jnp
from jax import lax
from jax.experimental import pallas as pl
from jax.experimental.pallas import tpu as pltpu
from jax.experimental.pallas import tpu_sc as plsc

N = 50000
E = 800000
H = 64
B = 512
NCLS = 7
NC = 2
NS = 16
NPAD = 50176
RPT = NPAD // NS
NBLK = NPAD // 128
CHUNK = 128
EPAD = 802816
CR = EPAD // CHUNK
CPT = CR // (NC * NS)
CPS = CR // NS
PW = 80


def _sc_degree(dst_r, ones_rows, z16):
    K = 4
    G = CPT // K
    mesh = plsc.VectorSubcoreMesh(core_axis_name="c", subcore_axis_name="s")

    @functools.partial(
        pl.kernel,
        out_type=jax.ShapeDtypeStruct((NC, NPAD, 16), jnp.float32),
        mesh=mesh,
        compiler_params=pltpu.CompilerParams(use_tc_tiling_on_sc=False),
        scratch_types=[
            pltpu.VMEM((K, CHUNK), jnp.int32),
            pltpu.VMEM((CHUNK, 16), jnp.float32),
            pltpu.VMEM_SHARED((NPAD, 16), jnp.float32),
        ],
    )
    def k(dst_h, ones_h, z_h, out_h, idx_v, rows_v, acc):
        c = lax.axis_index("c")
        s = lax.axis_index("s")
        pltpu.sync_copy(z_h.at[pl.ds(s * RPT, RPT)], acc.at[pl.ds(s * RPT, RPT)])
        pltpu.sync_copy(ones_h, rows_v)
        plsc.subcore_barrier()
        base = (c * NS + s) * CPT

        def body(g, carry):
            row0 = base + g * K
            pltpu.sync_copy(dst_h.at[pl.ds(row0, K)], idx_v)
            for j in range(K):
                pltpu.sync_copy(rows_v, acc.at[idx_v.at[j]], add=True)
            return carry

        lax.fori_loop(0, G, body, 0)
        plsc.subcore_barrier()
        pltpu.sync_copy(acc.at[pl.ds(s * RPT, RPT)],
                        out_h.at[c, pl.ds(s * RPT, RPT)])

    return k(dst_r, ones_rows, z16)


def _sc_agg16(src_r, dst_r, table, z16):
    K = 4
    HK = K // 2
    G = CPT // K
    mesh = plsc.VectorSubcoreMesh(core_axis_name="c", subcore_axis_name="s")

    @functools.partial(
        pl.kernel,
        out_type=jax.ShapeDtypeStruct((NC, NPAD, 16), jnp.float32),
        mesh=mesh,
        compiler_params=pltpu.CompilerParams(use_tc_tiling_on_sc=False),
        scratch_types=[
            pltpu.VMEM((K, CHUNK), jnp.int32),
            pltpu.VMEM((K, CHUNK), jnp.int32),
            pltpu.VMEM((HK, CHUNK, 16), jnp.float32),
            pltpu.VMEM((HK, CHUNK, 16), jnp.float32),
            pltpu.SemaphoreType.DMA,
            pltpu.VMEM_SHARED((NPAD, 16), jnp.float32),
        ],
    )
    def k(src_h, dst_h, tab_h, z_h, out_h,
          isv, idv, rows_a, rows_b, sem, acc):
        c = lax.axis_index("c")
        s = lax.axis_index("s")
        pltpu.sync_copy(z_h.at[pl.ds(s * RPT, RPT)], acc.at[pl.ds(s * RPT, RPT)])
        plsc.subcore_barrier()
        base = (c * NS + s) * CPT

        def body(g, carry):
            row0 = base + g * K
            pltpu.sync_copy(src_h.at[pl.ds(row0, K)], isv)
            pltpu.sync_copy(dst_h.at[pl.ds(row0, K)], idv)
            da = [pltpu.async_copy(tab_h.at[isv.at[j]], rows_a.at[j], sem)
                  for j in range(HK)]
            for d in da:
                d.wait()
            db = [pltpu.async_copy(tab_h.at[isv.at[HK + j]], rows_b.at[j], sem)
                  for j in range(HK)]
            for j in range(HK):
                pltpu.sync_copy(rows_a.at[j], acc.at[idv.at[j]], add=True)
            for d in db:
                d.wait()
            for j in range(HK):
                pltpu.sync_copy(rows_b.at[j], acc.at[idv.at[HK + j]], add=True)
            return carry

        lax.fori_loop(0, G, body, 0)
        plsc.subcore_barrier()
        pltpu.sync_copy(acc.at[pl.ds(s * RPT, RPT)],
                        out_h.at[c, pl.ds(s * RPT, RPT)])

    return k(src_r, dst_r, table, z16)


def _sc_agg32(srcs2, dst, y1cat, z32):
    ept = E // NS
    nfull = ept // CHUNK
    tail = ept - nfull * CHUNK
    mesh = plsc.VectorSubcoreMesh(core_axis_name="c", subcore_axis_name="s")

    @functools.partial(
        pl.kernel,
        out_type=jax.ShapeDtypeStruct((NC, NPAD, 32), jnp.float32),
        mesh=mesh,
        compiler_params=pltpu.CompilerParams(use_tc_tiling_on_sc=False),
        scratch_types=[
            pltpu.VMEM((CHUNK,), jnp.int32),
            pltpu.VMEM((CHUNK,), jnp.int32),
            pltpu.VMEM((tail,), jnp.int32),
            pltpu.VMEM((tail,), jnp.int32),
            pltpu.VMEM((CHUNK, 32), jnp.float32),
            pltpu.VMEM((tail, 32), jnp.float32),
            pltpu.VMEM_SHARED((NPAD, 32), jnp.float32),
        ],
    )
    def k(srcs_h, dst_h, tab_h, z_h, out_h,
          isv, idv, istv, idtv, rows_v, rowst_v, acc):
        c = lax.axis_index("c")
        s = lax.axis_index("s")
        pltpu.sync_copy(z_h.at[pl.ds(s * RPT, RPT)], acc.at[pl.ds(s * RPT, RPT)])
        plsc.subcore_barrier()
        base = s * ept

        def body(g, carry):
            off = base + g * CHUNK
            pltpu.sync_copy(srcs_h.at[c, pl.ds(off, CHUNK)], isv)
            pltpu.sync_copy(dst_h.at[pl.ds(off, CHUNK)], idv)
            pltpu.sync_copy(tab_h.at[isv], rows_v)
            pltpu.sync_copy(rows_v, acc.at[idv], add=True)
            return carry

        lax.fori_loop(0, nfull, body, 0)
        off = base + nfull * CHUNK
        pltpu.sync_copy(srcs_h.at[c, pl.ds(off, tail)], istv)
        pltpu.sync_copy(dst_h.at[pl.ds(off, tail)], idtv)
        pltpu.sync_copy(tab_h.at[istv], rowst_v)
        pltpu.sync_copy(rowst_v, acc.at[idtv], add=True)
        plsc.subcore_barrier()
        pltpu.sync_copy(acc.at[pl.ds(s * RPT, RPT)],
                        out_h.at[c, pl.ds(s * RPT, RPT)])

    return k(srcs2, dst, y1cat, z32)


def _sc_pool(h2pad, seg, z80):
    rpc = NPAD // NC
    rpt = rpc // NS
    nfull = rpt // CHUNK
    tail = rpt - nfull * CHUNK
    arows = (2 * B) // NS
    mesh = plsc.VectorSubcoreMesh(core_axis_name="c", subcore_axis_name="s")

    @functools.partial(
        pl.kernel,
        out_type=jax.ShapeDtypeStruct((NC, 2 * B, PW), jnp.float32),
        mesh=mesh,
        compiler_params=pltpu.CompilerParams(use_tc_tiling_on_sc=False),
        scratch_types=[
            pltpu.VMEM((CHUNK,), jnp.int32),
            pltpu.VMEM((tail,), jnp.int32),
            pltpu.VMEM((CHUNK, PW), jnp.float32),
            pltpu.VMEM((tail, PW), jnp.float32),
            pltpu.VMEM_SHARED((2 * B, PW), jnp.float32),
        ],
    )
    def k(h_h, seg_h, z_h, out_h, idx_v, idxt_v, rows_v, rowst_v, acc):
        c = lax.axis_index("c")
        s = lax.axis_index("s")
        pltpu.sync_copy(z_h.at[pl.ds(s * arows, arows)],
                        acc.at[pl.ds(s * arows, arows)])
        plsc.subcore_barrier()
        base = c * rpc + s * rpt

        def body(g, carry):
            off = base + g * CHUNK
            pltpu.sync_copy(seg_h.at[pl.ds(off, CHUNK)], idx_v)
            pltpu.sync_copy(h_h.at[pl.ds(off, CHUNK)], rows_v)
            pltpu.sync_copy(rows_v, acc.at[idx_v], add=True)
            return carry

        lax.fori_loop(0, nfull, body, 0)
        off = base + nfull * CHUNK
        pltpu.sync_copy(seg_h.at[pl.ds(off, tail)], idxt_v)
        pltpu.sync_copy(h_h.at[pl.ds(off, tail)], rowst_v)
        pltpu.sync_copy(rowst_v, acc.at[idxt_v], add=True)
        plsc.subcore_barrier()
        pltpu.sync_copy(acc.at[pl.ds(s * arows, arows)],
                        out_h.at[c, pl.ds(s * arows, arows)])

    return k(h2pad, seg, z80)


def _tc_prep(degp, x_pad, batch2d, gid2d):
    def body(degp_ref, x_ref, b_ref, g_ref, dinv_ref, y0_ref, seg_ref):
        deg16 = degp_ref[0] + degp_ref[1] + 1.0
        dinv16 = lax.rsqrt(deg16)
        dinv_ref[...] = dinv16
        y0 = x_ref[...] * dinv16[:, 0:2]
        y0_ref[...] = jnp.concatenate(
            [y0, jnp.zeros((128, 14), jnp.float32)], axis=1)
        seg_ref[...] = b_ref[...] + B * g_ref[...]

    return pl.pallas_call(
        body,
        grid=(NBLK,),
        in_specs=[
            pl.BlockSpec((NC, 128, 16), lambda i: (0, i, 0)),
            pl.BlockSpec((128, 2), lambda i: (i, 0)),
            pl.BlockSpec((1, 1, 128), lambda i: (i, 0, 0)),
            pl.BlockSpec((1, 1, 128), lambda i: (i, 0, 0)),
        ],
        out_specs=[
            pl.BlockSpec((128, 16), lambda i: (i, 0)),
            pl.BlockSpec((128, 16), lambda i: (i, 0)),
            pl.BlockSpec((1, 1, 128), lambda i: (i, 0, 0)),
        ],
        out_shape=[
            jax.ShapeDtypeStruct((NPAD, 16), jnp.float32),
            jax.ShapeDtypeStruct((NPAD, 16), jnp.float32),
            jax.ShapeDtypeStruct((NBLK, 1, 128), jnp.int32),
        ],
    )(degp, x_pad, batch2d, gid2d)


def _tc_layer1(s1p, y0pad, dinv16, W1, b1):
    def body(s1_ref, y0_ref, dv_ref, w1_ref, b1_ref, y1_ref):
        dinv16 = dv_ref[...]
        a1 = dinv16 * (s1_ref[0] + s1_ref[1] + y0_ref[...])
        h1 = a1[:, 0:1] * w1_ref[0:1, :] + a1[:, 1:2] * w1_ref[1:2, :]
        h1 = jnp.maximum(h1 + b1_ref[...], 0.0)
        y1 = dinv16[:, 0:1] * h1
        y1_ref[0] = y1[:, 0:32]
        y1_ref[1] = y1[:, 32:64]

    return pl.pallas_call(
        body,
        grid=(NBLK,),
        in_specs=[
            pl.BlockSpec((NC, 128, 16), lambda i: (0, i, 0)),
            pl.BlockSpec((128, 16), lambda i: (i, 0)),
            pl.BlockSpec((128, 16), lambda i: (i, 0)),
            pl.BlockSpec((2, H), lambda i: (0, 0)),
            pl.BlockSpec((1, H), lambda i: (0, 0)),
        ],
        out_specs=pl.BlockSpec((NC, 128, 32), lambda i: (0, i, 0)),
        out_shape=jax.ShapeDtypeStruct((NC, NPAD, 32), jnp.float32),
    )(s1p, y0pad, dinv16, W1, b1)


def _tc_layer2(s2p, y1p, dinv16, W2, b2):
    def body(s2_ref, y1_ref, dv_ref, w2_ref, b2_ref, out_ref):
        i = pl.program_id(0)
        s2 = jnp.concatenate([s2_ref[0] + y1_ref[0], s2_ref[1] + y1_ref[1]],
                             axis=1)
        a2 = dv_ref[...][:, 0:1] * s2
        h2 = jnp.dot(a2, w2_ref[...], preferred_element_type=jnp.float32)
        h2 = jnp.maximum(h2 + b2_ref[...], 0.0)
        row = i * 128 + lax.broadcasted_iota(jnp.int32, (128, 1), 0)
        valid = (row < N).astype(jnp.float32)
        out_ref[...] = jnp.concatenate(
            [h2 * valid, valid, jnp.zeros((128, PW - H - 1), jnp.float32)],
            axis=1)

    return pl.pallas_call(
        body,
        grid=(NBLK,),
        in_specs=[
            pl.BlockSpec((NC, 128, 32), lambda i: (0, i, 0)),
            pl.BlockSpec((NC, 128, 32), lambda i: (0, i, 0)),
            pl.BlockSpec((128, 16), lambda i: (i, 0)),
            pl.BlockSpec((H, H), lambda i: (0, 0)),
            pl.BlockSpec((1, H), lambda i: (0, 0)),
        ],
        out_specs=pl.BlockSpec((128, PW), lambda i: (i, 0)),
        out_shape=jax.ShapeDtypeStruct((NPAD, PW), jnp.float32),
    )(s2p, y1p, dinv16, W2, b2)


def _tc_final(poolp, Wfc, bfc):
    def body(p_ref, w_ref, b_ref, out_ref):
        P = p_ref[0] + p_ref[1]
        sums = P[:, 0:H]
        cnt = P[:, H:H + 1]
        p = sums / jnp.maximum(cnt, 1.0)
        combined = jnp.concatenate([p[0:B], p[B:2 * B]], axis=1)
        out_ref[...] = jnp.dot(combined, w_ref[...],
                               preferred_element_type=jnp.float32) + b_ref[...]

    return pl.pallas_call(
        body,
        grid=(1,),
        in_specs=[
            pl.BlockSpec((NC, 2 * B, PW), lambda i: (0, 0, 0)),
            pl.BlockSpec((2 * H, NCLS), lambda i: (0, 0)),
            pl.BlockSpec((1, NCLS), lambda i: (0, 0)),
        ],
        out_specs=pl.BlockSpec((B, NCLS), lambda i: (0, 0)),
        out_shape=jax.ShapeDtypeStruct((B, NCLS), jnp.float32),
    )(poolp, Wfc, bfc)


def kernel(x, edge_index, graph_id, batch, W1, b1, W2, b2, Wfc, bfc):
    src = edge_index[0].astype(jnp.int32)
    dst = edge_index[1].astype(jnp.int32)
    src_r = jnp.concatenate(
        [src, jnp.zeros((EPAD - E,), jnp.int32)]).reshape(CR, CHUNK)
    dst_r = jnp.concatenate(
        [dst, jnp.full((EPAD - E,), N, jnp.int32)]).reshape(CR, CHUNK)
    x_pad = jnp.pad(x, ((0, NPAD - N), (0, 0)))
    batch2d = jnp.pad(batch, (0, NPAD - N)).reshape(NBLK, 1, 128)
    gid2d = jnp.pad(graph_id, (0, NPAD - N)).reshape(NBLK, 1, 128)
    ones_rows = jnp.ones((CHUNK, 16), jnp.float32)
    z16 = jnp.zeros((NPAD, 16), jnp.float32)
    z32 = jnp.zeros((NPAD, 32), jnp.float32)
    z80 = jnp.zeros((2 * B, PW), jnp.float32)

    degp = _sc_degree(dst_r, ones_rows, z16)
    dinv16, y0pad, seg2d = _tc_prep(degp, x_pad, batch2d, gid2d)
    s1p = _sc_agg16(src_r, dst_r, y0pad, z16)
    y1p = _tc_layer1(s1p, y0pad, dinv16, W1, b1.reshape(1, H))
    srcs2 = jnp.stack([src, src + NPAD])
    y1cat = y1p.reshape(NC * NPAD, 32)
    s2p = _sc_agg32(srcs2, dst, y1cat, z32)
    h2pad = _tc_layer2(s2p, y1p, dinv16, W2, b2.reshape(1, H))
    poolp = _sc_pool(h2pad, seg2d.reshape(NPAD), z80)
    return _tc_final(poolp, Wfc, bfc.reshape(1, NCLS))


# speedup vs baseline: 15.7621x; 1.2459x over previous
import functools

import jax
import jax.numpy as jnp
from jax import lax
from jax.experimental import pallas as pl
from jax.experimental.pallas import tpu as pltpu
from jax.experimental.pallas import tpu_sc as plsc

N = 50000
E = 800000
H = 64
B = 512
NCLS = 7
NC = 2
NS = 16
NPAD = 50176
RPT = NPAD // NS
NBLK = NPAD // 128
CHUNK = 128
EPAD = 802816
CR = EPAD // CHUNK
CPT = CR // (NC * NS)
CPS = CR // NS
PW = 80


def _sc_degree(dst_r, ones_rows, z16):
    K = 4
    G = CPT // K
    mesh = plsc.VectorSubcoreMesh(core_axis_name="c", subcore_axis_name="s")

    @functools.partial(
        pl.kernel,
        out_type=jax.ShapeDtypeStruct((NC, NPAD, 16), jnp.float32),
        mesh=mesh,
        compiler_params=pltpu.CompilerParams(use_tc_tiling_on_sc=False),
        scratch_types=[
            pltpu.VMEM((K, CHUNK), jnp.int32),
            pltpu.VMEM((CHUNK, 16), jnp.float32),
            pltpu.VMEM_SHARED((NPAD, 16), jnp.float32),
        ],
    )
    def k(dst_h, ones_h, z_h, out_h, idx_v, rows_v, acc):
        c = lax.axis_index("c")
        s = lax.axis_index("s")
        pltpu.sync_copy(z_h.at[pl.ds(s * RPT, RPT)], acc.at[pl.ds(s * RPT, RPT)])
        pltpu.sync_copy(ones_h, rows_v)
        plsc.subcore_barrier()
        base = (c * NS + s) * CPT

        def body(g, carry):
            row0 = base + g * K
            pltpu.sync_copy(dst_h.at[pl.ds(row0, K)], idx_v)
            for j in range(K):
                pltpu.sync_copy(rows_v, acc.at[idx_v.at[j]], add=True)
            return carry

        lax.fori_loop(0, G, body, 0)
        plsc.subcore_barrier()
        pltpu.sync_copy(acc.at[pl.ds(s * RPT, RPT)],
                        out_h.at[c, pl.ds(s * RPT, RPT)])

    return k(dst_r, ones_rows, z16)


def _sc_agg16(src_r, dst_r, table, z16):
    K = 4
    HK = K // 2
    G = CPT // K
    mesh = plsc.VectorSubcoreMesh(core_axis_name="c", subcore_axis_name="s")

    @functools.partial(
        pl.kernel,
        out_type=jax.ShapeDtypeStruct((NC, NPAD, 16), jnp.float32),
        mesh=mesh,
        compiler_params=pltpu.CompilerParams(use_tc_tiling_on_sc=False),
        scratch_types=[
            pltpu.VMEM((K, CHUNK), jnp.int32),
            pltpu.VMEM((K, CHUNK), jnp.int32),
            pltpu.VMEM((HK, CHUNK, 16), jnp.float32),
            pltpu.VMEM((HK, CHUNK, 16), jnp.float32),
            pltpu.SemaphoreType.DMA,
            pltpu.VMEM_SHARED((NPAD, 16), jnp.float32),
        ],
    )
    def k(src_h, dst_h, tab_h, z_h, out_h,
          isv, idv, rows_a, rows_b, sem, acc):
        c = lax.axis_index("c")
        s = lax.axis_index("s")
        pltpu.sync_copy(z_h.at[pl.ds(s * RPT, RPT)], acc.at[pl.ds(s * RPT, RPT)])
        plsc.subcore_barrier()
        base = (c * NS + s) * CPT

        def body(g, carry):
            row0 = base + g * K
            pltpu.sync_copy(src_h.at[pl.ds(row0, K)], isv)
            pltpu.sync_copy(dst_h.at[pl.ds(row0, K)], idv)
            da = [pltpu.async_copy(tab_h.at[isv.at[j]], rows_a.at[j], sem)
                  for j in range(HK)]
            for d in da:
                d.wait()
            db = [pltpu.async_copy(tab_h.at[isv.at[HK + j]], rows_b.at[j], sem)
                  for j in range(HK)]
            for j in range(HK):
                pltpu.sync_copy(rows_a.at[j], acc.at[idv.at[j]], add=True)
            for d in db:
                d.wait()
            for j in range(HK):
                pltpu.sync_copy(rows_b.at[j], acc.at[idv.at[HK + j]], add=True)
            return carry

        lax.fori_loop(0, G, body, 0)
        plsc.subcore_barrier()
        pltpu.sync_copy(acc.at[pl.ds(s * RPT, RPT)],
                        out_h.at[c, pl.ds(s * RPT, RPT)])

    return k(src_r, dst_r, table, z16)


def _sc_agg32(src_r, dst_r, y1p, z32):
    K = 4
    HK = K // 2
    G = CPS // K
    mesh = plsc.VectorSubcoreMesh(core_axis_name="c", subcore_axis_name="s")

    @functools.partial(
        pl.kernel,
        out_type=jax.ShapeDtypeStruct((NC, NPAD, 32), jnp.float32),
        mesh=mesh,
        compiler_params=pltpu.CompilerParams(use_tc_tiling_on_sc=False),
        scratch_types=[
            pltpu.VMEM((K, CHUNK), jnp.int32),
            pltpu.VMEM((K, CHUNK), jnp.int32),
            pltpu.VMEM((HK, CHUNK, 32), jnp.float32),
            pltpu.VMEM((HK, CHUNK, 32), jnp.float32),
            pltpu.SemaphoreType.DMA,
            pltpu.VMEM_SHARED((NPAD, 32), jnp.float32),
        ],
    )
    def k(src_h, dst_h, tab_h, z_h, out_h,
          isv, idv, rows_a, rows_b, sem, acc):
        c = lax.axis_index("c")
        s = lax.axis_index("s")
        pltpu.sync_copy(z_h.at[pl.ds(s * RPT, RPT)], acc.at[pl.ds(s * RPT, RPT)])
        plsc.subcore_barrier()
        base = s * CPS
        tab_c = tab_h.at[c]

        def body(g, carry):
            row0 = base + g * K
            pltpu.sync_copy(src_h.at[pl.ds(row0, K)], isv)
            pltpu.sync_copy(dst_h.at[pl.ds(row0, K)], idv)
            da = [pltpu.async_copy(tab_c.at[isv.at[j]], rows_a.at[j], sem)
                  for j in range(HK)]
            for d in da:
                d.wait()
            db = [pltpu.async_copy(tab_c.at[isv.at[HK + j]], rows_b.at[j], sem)
                  for j in range(HK)]
            for j in range(HK):
                pltpu.sync_copy(rows_a.at[j], acc.at[idv.at[j]], add=True)
            for d in db:
                d.wait()
            for j in range(HK):
                pltpu.sync_copy(rows_b.at[j], acc.at[idv.at[HK + j]], add=True)
            return carry

        lax.fori_loop(0, G, body, 0)
        plsc.subcore_barrier()
        pltpu.sync_copy(acc.at[pl.ds(s * RPT, RPT)],
                        out_h.at[c, pl.ds(s * RPT, RPT)])

    return k(src_r, dst_r, y1p, z32)


def _sc_pool(h2pad, seg, z80):
    rpc = NPAD // NC
    rpt = rpc // NS
    nfull = rpt // CHUNK
    tail = rpt - nfull * CHUNK
    arows = (2 * B) // NS
    mesh = plsc.VectorSubcoreMesh(core_axis_name="c", subcore_axis_name="s")

    @functools.partial(
        pl.kernel,
        out_type=jax.ShapeDtypeStruct((NC, 2 * B, PW), jnp.float32),
        mesh=mesh,
        compiler_params=pltpu.CompilerParams(use_tc_tiling_on_sc=False),
        scratch_types=[
            pltpu.VMEM((CHUNK,), jnp.int32),
            pltpu.VMEM((tail,), jnp.int32),
            pltpu.VMEM((CHUNK, PW), jnp.float32),
            pltpu.VMEM((tail, PW), jnp.float32),
            pltpu.VMEM_SHARED((2 * B, PW), jnp.float32),
        ],
    )
    def k(h_h, seg_h, z_h, out_h, idx_v, idxt_v, rows_v, rowst_v, acc):
        c = lax.axis_index("c")
        s = lax.axis_index("s")
        pltpu.sync_copy(z_h.at[pl.ds(s * arows, arows)],
                        acc.at[pl.ds(s * arows, arows)])
        plsc.subcore_barrier()
        base = c * rpc + s * rpt

        def body(g, carry):
            off = base + g * CHUNK
            pltpu.sync_copy(seg_h.at[pl.ds(off, CHUNK)], idx_v)
            pltpu.sync_copy(h_h.at[pl.ds(off, CHUNK)], rows_v)
            pltpu.sync_copy(rows_v, acc.at[idx_v], add=True)
            return carry

        lax.fori_loop(0, nfull, body, 0)
        off = base + nfull * CHUNK
        pltpu.sync_copy(seg_h.at[pl.ds(off, tail)], idxt_v)
        pltpu.sync_copy(h_h.at[pl.ds(off, tail)], rowst_v)
        pltpu.sync_copy(rowst_v, acc.at[idxt_v], add=True)
        plsc.subcore_barrier()
        pltpu.sync_copy(acc.at[pl.ds(s * arows, arows)],
                        out_h.at[c, pl.ds(s * arows, arows)])

    return k(h2pad, seg, z80)


def _tc_prep(degp, x_pad, batch2d, gid2d):
    def body(degp_ref, x_ref, b_ref, g_ref, dinv_ref, y0_ref, seg_ref):
        deg16 = degp_ref[0] + degp_ref[1] + 1.0
        dinv16 = lax.rsqrt(deg16)
        dinv_ref[...] = dinv16
        y0 = x_ref[...] * dinv16[:, 0:2]
        y0_ref[...] = jnp.concatenate(
            [y0, jnp.zeros((128, 14), jnp.float32)], axis=1)
        seg_ref[...] = b_ref[...] + B * g_ref[...]

    return pl.pallas_call(
        body,
        grid=(NBLK,),
        in_specs=[
            pl.BlockSpec((NC, 128, 16), lambda i: (0, i, 0)),
            pl.BlockSpec((128, 2), lambda i: (i, 0)),
            pl.BlockSpec((1, 1, 128), lambda i: (i, 0, 0)),
            pl.BlockSpec((1, 1, 128), lambda i: (i, 0, 0)),
        ],
        out_specs=[
            pl.BlockSpec((128, 16), lambda i: (i, 0)),
            pl.BlockSpec((128, 16), lambda i: (i, 0)),
            pl.BlockSpec((1, 1, 128), lambda i: (i, 0, 0)),
        ],
        out_shape=[
            jax.ShapeDtypeStruct((NPAD, 16), jnp.float32),
            jax.ShapeDtypeStruct((NPAD, 16), jnp.float32),
            jax.ShapeDtypeStruct((NBLK, 1, 128), jnp.int32),
        ],
    )(degp, x_pad, batch2d, gid2d)


def _tc_layer1(s1p, y0pad, dinv16, W1, b1):
    def body(s1_ref, y0_ref, dv_ref, w1_ref, b1_ref, y1_ref):
        dinv16 = dv_ref[...]
        a1 = dinv16 * (s1_ref[0] + s1_ref[1] + y0_ref[...])
        h1 = a1[:, 0:1] * w1_ref[0:1, :] + a1[:, 1:2] * w1_ref[1:2, :]
        h1 = jnp.maximum(h1 + b1_ref[...], 0.0)
        y1 = dinv16[:, 0:1] * h1
        y1_ref[0] = y1[:, 0:32]
        y1_ref[1] = y1[:, 32:64]

    return pl.pallas_call(
        body,
        grid=(NBLK,),
        in_specs=[
            pl.BlockSpec((NC, 128, 16), lambda i: (0, i, 0)),
            pl.BlockSpec((128, 16), lambda i: (i, 0)),
            pl.BlockSpec((128, 16), lambda i: (i, 0)),
            pl.BlockSpec((2, H), lambda i: (0, 0)),
            pl.BlockSpec((1, H), lambda i: (0, 0)),
        ],
        out_specs=pl.BlockSpec((NC, 128, 32), lambda i: (0, i, 0)),
        out_shape=jax.ShapeDtypeStruct((NC, NPAD, 32), jnp.float32),
    )(s1p, y0pad, dinv16, W1, b1)


def _tc_layer2(s2p, y1p, dinv16, W2, b2):
    def body(s2_ref, y1_ref, dv_ref, w2_ref, b2_ref, out_ref):
        i = pl.program_id(0)
        s2 = jnp.concatenate([s2_ref[0] + y1_ref[0], s2_ref[1] + y1_ref[1]],
                             axis=1)
        a2 = dv_ref[...][:, 0:1] * s2
        h2 = jnp.dot(a2, w2_ref[...], preferred_element_type=jnp.float32)
        h2 = jnp.maximum(h2 + b2_ref[...], 0.0)
        row = i * 128 + lax.broadcasted_iota(jnp.int32, (128, 1), 0)
        valid = (row < N).astype(jnp.float32)
        out_ref[...] = jnp.concatenate(
            [h2 * valid, valid, jnp.zeros((128, PW - H - 1), jnp.float32)],
            axis=1)

    return pl.pallas_call(
        body,
        grid=(NBLK,),
        in_specs=[
            pl.BlockSpec((NC, 128, 32), lambda i: (0, i, 0)),
            pl.BlockSpec((NC, 128, 32), lambda i: (0, i, 0)),
            pl.BlockSpec((128, 16), lambda i: (i, 0)),
            pl.BlockSpec((H, H), lambda i: (0, 0)),
            pl.BlockSpec((1, H), lambda i: (0, 0)),
        ],
        out_specs=pl.BlockSpec((128, PW), lambda i: (i, 0)),
        out_shape=jax.ShapeDtypeStruct((NPAD, PW), jnp.float32),
    )(s2p, y1p, dinv16, W2, b2)


def _tc_final(poolp, Wfc, bfc):
    def body(p_ref, w_ref, b_ref, out_ref):
        P = p_ref[0] + p_ref[1]
        sums = P[:, 0:H]
        cnt = P[:, H:H + 1]
        p = sums / jnp.maximum(cnt, 1.0)
        combined = jnp.concatenate([p[0:B], p[B:2 * B]], axis=1)
        out_ref[...] = jnp.dot(combined, w_ref[...],
                               preferred_element_type=jnp.float32) + b_ref[...]

    return pl.pallas_call(
        body,
        grid=(1,),
        in_specs=[
            pl.BlockSpec((NC, 2 * B, PW), lambda i: (0, 0, 0)),
            pl.BlockSpec((2 * H, NCLS), lambda i: (0, 0)),
            pl.BlockSpec((1, NCLS), lambda i: (0, 0)),
        ],
        out_specs=pl.BlockSpec((B, NCLS), lambda i: (0, 0)),
        out_shape=jax.ShapeDtypeStruct((B, NCLS), jnp.float32),
    )(poolp, Wfc, bfc)


def kernel(x, edge_index, graph_id, batch, W1, b1, W2, b2, Wfc, bfc):
    src = edge_index[0].astype(jnp.int32)
    dst = edge_index[1].astype(jnp.int32)
    src_r = jnp.concatenate(
        [src, jnp.zeros((EPAD - E,), jnp.int32)]).reshape(CR, CHUNK)
    dst_r = jnp.concatenate(
        [dst, jnp.full((EPAD - E,), N, jnp.int32)]).reshape(CR, CHUNK)
    x_pad = jnp.pad(x, ((0, NPAD - N), (0, 0)))
    batch2d = jnp.pad(batch, (0, NPAD - N)).reshape(NBLK, 1, 128)
    gid2d = jnp.pad(graph_id, (0, NPAD - N)).reshape(NBLK, 1, 128)
    ones_rows = jnp.ones((CHUNK, 16), jnp.float32)
    z16 = jnp.zeros((NPAD, 16), jnp.float32)
    z32 = jnp.zeros((NPAD, 32), jnp.float32)
    z80 = jnp.zeros((2 * B, PW), jnp.float32)

    degp = _sc_degree(dst_r, ones_rows, z16)
    dinv16, y0pad, seg2d = _tc_prep(degp, x_pad, batch2d, gid2d)
    s1p = _sc_agg16(src_r, dst_r, y0pad, z16)
    y1p = _tc_layer1(s1p, y0pad, dinv16, W1, b1.reshape(1, H))
    s2p = _sc_agg32(src_r, dst_r, y1p, z32)
    h2pad = _tc_layer2(s2p, y1p, dinv16, W2, b2.reshape(1, H))
    poolp = _sc_pool(h2pad, seg2d.reshape(NPAD), z80)
    return _tc_final(poolp, Wfc, bfc.reshape(1, NCLS))


# speedup vs baseline: 23.5796x; 1.4960x over previous
import functools

import jax
import jax.numpy as jnp
from jax import lax
from jax.experimental import pallas as pl
from jax.experimental.pallas import tpu as pltpu
from jax.experimental.pallas import tpu_sc as plsc

N = 50000
E = 800000
H = 64
B = 512
NCLS = 7
NC = 2
NS = 16
NPAD = 50176
RPT = NPAD // NS
NBLK = NPAD // 128
BR = 1024
NB = NPAD // BR
CHUNK = 128
EPAD = 802816
CR = EPAD // CHUNK
CPT = CR // (NC * NS)
CPS = CR // NS
PW = 80


def _sc_degree(dst_r, ones_rows, z16):
    K = 4
    G = CPT // K
    mesh = plsc.VectorSubcoreMesh(core_axis_name="c", subcore_axis_name="s")

    @functools.partial(
        pl.kernel,
        out_type=jax.ShapeDtypeStruct((NC, NPAD, 16), jnp.float32),
        mesh=mesh,
        compiler_params=pltpu.CompilerParams(use_tc_tiling_on_sc=False),
        scratch_types=[
            pltpu.VMEM((K, CHUNK), jnp.int32),
            pltpu.VMEM((CHUNK, 16), jnp.float32),
            pltpu.VMEM_SHARED((NPAD, 16), jnp.float32),
        ],
    )
    def k(dst_h, ones_h, z_h, out_h, idx_v, rows_v, acc):
        c = lax.axis_index("c")
        s = lax.axis_index("s")
        pltpu.sync_copy(z_h.at[pl.ds(s * RPT, RPT)], acc.at[pl.ds(s * RPT, RPT)])
        pltpu.sync_copy(ones_h, rows_v)
        plsc.subcore_barrier()
        base = (c * NS + s) * CPT

        def body(g, carry):
            row0 = base + g * K
            pltpu.sync_copy(dst_h.at[pl.ds(row0, K)], idx_v)
            for j in range(K):
                pltpu.sync_copy(rows_v, acc.at[idx_v.at[j]], add=True)
            return carry

        lax.fori_loop(0, G, body, 0)
        plsc.subcore_barrier()
        pltpu.sync_copy(acc.at[pl.ds(s * RPT, RPT)],
                        out_h.at[c, pl.ds(s * RPT, RPT)])

    return k(dst_r, ones_rows, z16)


def _sc_agg16(src_r, dst_r, table, z16):
    K = 4
    HK = K // 2
    G = CPT // K
    mesh = plsc.VectorSubcoreMesh(core_axis_name="c", subcore_axis_name="s")

    @functools.partial(
        pl.kernel,
        out_type=jax.ShapeDtypeStruct((NC, NPAD, 16), jnp.float32),
        mesh=mesh,
        compiler_params=pltpu.CompilerParams(use_tc_tiling_on_sc=False),
        scratch_types=[
            pltpu.VMEM((K, CHUNK), jnp.int32),
            pltpu.VMEM((K, CHUNK), jnp.int32),
            pltpu.VMEM((HK, CHUNK, 16), jnp.float32),
            pltpu.VMEM((HK, CHUNK, 16), jnp.float32),
            pltpu.SemaphoreType.DMA,
            pltpu.VMEM_SHARED((NPAD, 16), jnp.float32),
        ],
    )
    def k(src_h, dst_h, tab_h, z_h, out_h,
          isv, idv, rows_a, rows_b, sem, acc):
        c = lax.axis_index("c")
        s = lax.axis_index("s")
        pltpu.sync_copy(z_h.at[pl.ds(s * RPT, RPT)], acc.at[pl.ds(s * RPT, RPT)])
        plsc.subcore_barrier()
        base = (c * NS + s) * CPT

        def body(g, carry):
            row0 = base + g * K
            pltpu.sync_copy(src_h.at[pl.ds(row0, K)], isv)
            pltpu.sync_copy(dst_h.at[pl.ds(row0, K)], idv)
            da = [pltpu.async_copy(tab_h.at[isv.at[j]], rows_a.at[j], sem)
                  for j in range(HK)]
            for d in da:
                d.wait()
            db = [pltpu.async_copy(tab_h.at[isv.at[HK + j]], rows_b.at[j], sem)
                  for j in range(HK)]
            for j in range(HK):
                pltpu.sync_copy(rows_a.at[j], acc.at[idv.at[j]], add=True)
            for d in db:
                d.wait()
            for j in range(HK):
                pltpu.sync_copy(rows_b.at[j], acc.at[idv.at[HK + j]], add=True)
            return carry

        lax.fori_loop(0, G, body, 0)
        plsc.subcore_barrier()
        pltpu.sync_copy(acc.at[pl.ds(s * RPT, RPT)],
                        out_h.at[c, pl.ds(s * RPT, RPT)])

    return k(src_r, dst_r, table, z16)


def _sc_agg32(src_r, dst_r, y1p, z32):
    K = 4
    HK = K // 2
    G = CPS // K
    mesh = plsc.VectorSubcoreMesh(core_axis_name="c", subcore_axis_name="s")

    @functools.partial(
        pl.kernel,
        out_type=jax.ShapeDtypeStruct((NC, NPAD, 32), jnp.float32),
        mesh=mesh,
        compiler_params=pltpu.CompilerParams(use_tc_tiling_on_sc=False),
        scratch_types=[
            pltpu.VMEM((K, CHUNK), jnp.int32),
            pltpu.VMEM((K, CHUNK), jnp.int32),
            pltpu.VMEM((HK, CHUNK, 32), jnp.float32),
            pltpu.VMEM((HK, CHUNK, 32), jnp.float32),
            pltpu.SemaphoreType.DMA,
            pltpu.VMEM_SHARED((NPAD, 32), jnp.float32),
        ],
    )
    def k(src_h, dst_h, tab_h, z_h, out_h,
          isv, idv, rows_a, rows_b, sem, acc):
        c = lax.axis_index("c")
        s = lax.axis_index("s")
        pltpu.sync_copy(z_h.at[pl.ds(s * RPT, RPT)], acc.at[pl.ds(s * RPT, RPT)])
        plsc.subcore_barrier()
        base = s * CPS
        tab_c = tab_h.at[c]

        def body(g, carry):
            row0 = base + g * K
            pltpu.sync_copy(src_h.at[pl.ds(row0, K)], isv)
            pltpu.sync_copy(dst_h.at[pl.ds(row0, K)], idv)
            da = [pltpu.async_copy(tab_c.at[isv.at[j]], rows_a.at[j], sem)
                  for j in range(HK)]
            for d in da:
                d.wait()
            db = [pltpu.async_copy(tab_c.at[isv.at[HK + j]], rows_b.at[j], sem)
                  for j in range(HK)]
            for j in range(HK):
                pltpu.sync_copy(rows_a.at[j], acc.at[idv.at[j]], add=True)
            for d in db:
                d.wait()
            for j in range(HK):
                pltpu.sync_copy(rows_b.at[j], acc.at[idv.at[HK + j]], add=True)
            return carry

        lax.fori_loop(0, G, body, 0)
        plsc.subcore_barrier()
        pltpu.sync_copy(acc.at[pl.ds(s * RPT, RPT)],
                        out_h.at[c, pl.ds(s * RPT, RPT)])

    return k(src_r, dst_r, y1p, z32)


def _sc_pool(h2pad, seg, z80):
    rpc = NPAD // NC
    rpt = rpc // NS
    nfull = rpt // CHUNK
    tail = rpt - nfull * CHUNK
    arows = (2 * B) // NS
    mesh = plsc.VectorSubcoreMesh(core_axis_name="c", subcore_axis_name="s")

    @functools.partial(
        pl.kernel,
        out_type=jax.ShapeDtypeStruct((NC, 2 * B, PW), jnp.float32),
        mesh=mesh,
        compiler_params=pltpu.CompilerParams(use_tc_tiling_on_sc=False),
        scratch_types=[
            pltpu.VMEM((CHUNK,), jnp.int32),
            pltpu.VMEM((tail,), jnp.int32),
            pltpu.VMEM((CHUNK, PW), jnp.float32),
            pltpu.VMEM((tail, PW), jnp.float32),
            pltpu.VMEM_SHARED((2 * B, PW), jnp.float32),
        ],
    )
    def k(h_h, seg_h, z_h, out_h, idx_v, idxt_v, rows_v, rowst_v, acc):
        c = lax.axis_index("c")
        s = lax.axis_index("s")
        pltpu.sync_copy(z_h.at[pl.ds(s * arows, arows)],
                        acc.at[pl.ds(s * arows, arows)])
        plsc.subcore_barrier()
        base = c * rpc + s * rpt

        def body(g, carry):
            off = base + g * CHUNK
            pltpu.sync_copy(seg_h.at[pl.ds(off, CHUNK)], idx_v)
            pltpu.sync_copy(h_h.at[pl.ds(off, CHUNK)], rows_v)
            pltpu.sync_copy(rows_v, acc.at[idx_v], add=True)
            return carry

        lax.fori_loop(0, nfull, body, 0)
        off = base + nfull * CHUNK
        pltpu.sync_copy(seg_h.at[pl.ds(off, tail)], idxt_v)
        pltpu.sync_copy(h_h.at[pl.ds(off, tail)], rowst_v)
        pltpu.sync_copy(rowst_v, acc.at[idxt_v], add=True)
        plsc.subcore_barrier()
        pltpu.sync_copy(acc.at[pl.ds(s * arows, arows)],
                        out_h.at[c, pl.ds(s * arows, arows)])

    return k(h2pad, seg, z80)


def _tc_prep(degp, x_pad, batch2d, gid2d):
    def body(degp_ref, x_ref, b_ref, g_ref, dinv_ref, y0_ref, seg_ref):
        deg16 = degp_ref[0] + degp_ref[1] + 1.0
        dinv16 = lax.rsqrt(deg16)
        dinv_ref[...] = dinv16
        y0 = x_ref[...] * dinv16[:, 0:2]
        y0_ref[...] = jnp.concatenate(
            [y0, jnp.zeros((BR, 14), jnp.float32)], axis=1)
        seg_ref[...] = b_ref[...] + B * g_ref[...]

    return pl.pallas_call(
        body,
        grid=(NB,),
        in_specs=[
            pl.BlockSpec((NC, BR, 16), lambda i: (0, i, 0)),
            pl.BlockSpec((BR, 2), lambda i: (i, 0)),
            pl.BlockSpec((1, 1, BR), lambda i: (i, 0, 0)),
            pl.BlockSpec((1, 1, BR), lambda i: (i, 0, 0)),
        ],
        out_specs=[
            pl.BlockSpec((BR, 16), lambda i: (i, 0)),
            pl.BlockSpec((BR, 16), lambda i: (i, 0)),
            pl.BlockSpec((1, 1, BR), lambda i: (i, 0, 0)),
        ],
        out_shape=[
            jax.ShapeDtypeStruct((NPAD, 16), jnp.float32),
            jax.ShapeDtypeStruct((NPAD, 16), jnp.float32),
            jax.ShapeDtypeStruct((NB, 1, BR), jnp.int32),
        ],
    )(degp, x_pad, batch2d, gid2d)


def _tc_layer1(s1p, y0pad, dinv16, W1, b1):
    def body(s1_ref, y0_ref, dv_ref, w1_ref, b1_ref, y1_ref):
        dinv16 = dv_ref[...]
        a1 = dinv16 * (s1_ref[0] + s1_ref[1] + y0_ref[...])
        h1 = a1[:, 0:1] * w1_ref[0:1, :] + a1[:, 1:2] * w1_ref[1:2, :]
        h1 = jnp.maximum(h1 + b1_ref[...], 0.0)
        y1 = dinv16[:, 0:1] * h1
        y1_ref[0] = y1[:, 0:32]
        y1_ref[1] = y1[:, 32:64]

    return pl.pallas_call(
        body,
        grid=(NB,),
        in_specs=[
            pl.BlockSpec((NC, BR, 16), lambda i: (0, i, 0)),
            pl.BlockSpec((BR, 16), lambda i: (i, 0)),
            pl.BlockSpec((BR, 16), lambda i: (i, 0)),
            pl.BlockSpec((2, H), lambda i: (0, 0)),
            pl.BlockSpec((1, H), lambda i: (0, 0)),
        ],
        out_specs=pl.BlockSpec((NC, BR, 32), lambda i: (0, i, 0)),
        out_shape=jax.ShapeDtypeStruct((NC, NPAD, 32), jnp.float32),
    )(s1p, y0pad, dinv16, W1, b1)


def _tc_layer2(s2p, y1p, dinv16, W2, b2):
    def body(s2_ref, y1_ref, dv_ref, w2_ref, b2_ref, out_ref):
        i = pl.program_id(0)
        s2 = jnp.concatenate([s2_ref[0] + y1_ref[0], s2_ref[1] + y1_ref[1]],
                             axis=1)
        a2 = dv_ref[...][:, 0:1] * s2
        h2 = jnp.dot(a2, w2_ref[...], preferred_element_type=jnp.float32)
        h2 = jnp.maximum(h2 + b2_ref[...], 0.0)
        row = i * BR + lax.broadcasted_iota(jnp.int32, (BR, 1), 0)
        valid = (row < N).astype(jnp.float32)
        out_ref[...] = jnp.concatenate(
            [h2 * valid, valid, jnp.zeros((BR, PW - H - 1), jnp.float32)],
            axis=1)

    return pl.pallas_call(
        body,
        grid=(NB,),
        in_specs=[
            pl.BlockSpec((NC, BR, 32), lambda i: (0, i, 0)),
            pl.BlockSpec((NC, BR, 32), lambda i: (0, i, 0)),
            pl.BlockSpec((BR, 16), lambda i: (i, 0)),
            pl.BlockSpec((H, H), lambda i: (0, 0)),
            pl.BlockSpec((1, H), lambda i: (0, 0)),
        ],
        out_specs=pl.BlockSpec((BR, PW), lambda i: (i, 0)),
        out_shape=jax.ShapeDtypeStruct((NPAD, PW), jnp.float32),
    )(s2p, y1p, dinv16, W2, b2)


def _tc_final(poolp, Wfc, bfc):
    def body(p_ref, w_ref, b_ref, out_ref):
        P = p_ref[0] + p_ref[1]
        sums = P[:, 0:H]
        cnt = P[:, H:H + 1]
        p = sums / jnp.maximum(cnt, 1.0)
        combined = jnp.concatenate([p[0:B], p[B:2 * B]], axis=1)
        out_ref[...] = jnp.dot(combined, w_ref[...],
                               preferred_element_type=jnp.float32) + b_ref[...]

    return pl.pallas_call(
        body,
        grid=(1,),
        in_specs=[
            pl.BlockSpec((NC, 2 * B, PW), lambda i: (0, 0, 0)),
            pl.BlockSpec((2 * H, NCLS), lambda i: (0, 0)),
            pl.BlockSpec((1, NCLS), lambda i: (0, 0)),
        ],
        out_specs=pl.BlockSpec((B, NCLS), lambda i: (0, 0)),
        out_shape=jax.ShapeDtypeStruct((B, NCLS), jnp.float32),
    )(poolp, Wfc, bfc)


def kernel(x, edge_index, graph_id, batch, W1, b1, W2, b2, Wfc, bfc):
    src = edge_index[0].astype(jnp.int32)
    dst = edge_index[1].astype(jnp.int32)
    src_r = jnp.concatenate(
        [src, jnp.zeros((EPAD - E,), jnp.int32)]).reshape(CR, CHUNK)
    dst_r = jnp.concatenate(
        [dst, jnp.full((EPAD - E,), N, jnp.int32)]).reshape(CR, CHUNK)
    x_pad = jnp.pad(x, ((0, NPAD - N), (0, 0)))
    batch2d = jnp.pad(batch, (0, NPAD - N)).reshape(NB, 1, BR)
    gid2d = jnp.pad(graph_id, (0, NPAD - N)).reshape(NB, 1, BR)
    ones_rows = jnp.ones((CHUNK, 16), jnp.float32)
    z16 = jnp.zeros((NPAD, 16), jnp.float32)
    z32 = jnp.zeros((NPAD, 32), jnp.float32)
    z80 = jnp.zeros((2 * B, PW), jnp.float32)

    degp = _sc_degree(dst_r, ones_rows, z16)
    dinv16, y0pad, seg2d = _tc_prep(degp, x_pad, batch2d, gid2d)
    s1p = _sc_agg16(src_r, dst_r, y0pad, z16)
    y1p = _tc_layer1(s1p, y0pad, dinv16, W1, b1.reshape(1, H))
    s2p = _sc_agg32(src_r, dst_r, y1p, z32)
    h2pad = _tc_layer2(s2p, y1p, dinv16, W2, b2.reshape(1, H))
    poolp = _sc_pool(h2pad, seg2d.reshape(NPAD), z80)
    return _tc_final(poolp, Wfc, bfc.reshape(1, NCLS))


# speedup vs baseline: 27.6374x; 1.1721x over previous
import functools

import jax
import jax.numpy as jnp
from jax import lax
from jax.experimental import pallas as pl
from jax.experimental.pallas import tpu as pltpu
from jax.experimental.pallas import tpu_sc as plsc

N = 50000
E = 800000
H = 64
B = 512
NCLS = 7
NC = 2
NS = 16
NPAD = 50176
RPT = NPAD // NS
NBLK = NPAD // 128
BR = 1024
NB = NPAD // BR
CHUNK = 128
EPAD = 802816
CR = EPAD // CHUNK
CPT = CR // (NC * NS)
CPS = CR // NS
PW = 80


def _sc_degree(dst_r, ones_rows, z16):
    K = 7
    G = CPT // K
    mesh = plsc.VectorSubcoreMesh(core_axis_name="c", subcore_axis_name="s")

    @functools.partial(
        pl.kernel,
        out_type=jax.ShapeDtypeStruct((NC, NPAD, 16), jnp.float32),
        mesh=mesh,
        compiler_params=pltpu.CompilerParams(use_tc_tiling_on_sc=False),
        scratch_types=[
            pltpu.VMEM((K, CHUNK), jnp.int32),
            pltpu.VMEM((CHUNK, 16), jnp.float32),
            pltpu.SemaphoreType.DMA,
            pltpu.VMEM_SHARED((NPAD, 16), jnp.float32),
        ],
    )
    def k(dst_h, ones_h, z_h, out_h, idx_v, rows_v, sem, acc):
        c = lax.axis_index("c")
        s = lax.axis_index("s")
        pltpu.sync_copy(z_h.at[pl.ds(s * RPT, RPT)], acc.at[pl.ds(s * RPT, RPT)])
        pltpu.sync_copy(ones_h, rows_v)
        plsc.subcore_barrier()
        base = (c * NS + s) * CPT

        def body(g, carry):
            row0 = base + g * K
            pltpu.sync_copy(dst_h.at[pl.ds(row0, K)], idx_v)
            ds = [pltpu.async_copy(rows_v, acc.at[idx_v.at[j]], sem, add=True)
                  for j in range(K)]
            for d in ds:
                d.wait()
            return carry

        lax.fori_loop(0, G, body, 0)
        plsc.subcore_barrier()
        pltpu.sync_copy(acc.at[pl.ds(s * RPT, RPT)],
                        out_h.at[c, pl.ds(s * RPT, RPT)])

    return k(dst_r, ones_rows, z16)


def _sc_agg16(src_r, dst_r, table, z16):
    K = 7
    G = CPT // K
    mesh = plsc.VectorSubcoreMesh(core_axis_name="c", subcore_axis_name="s")

    @functools.partial(
        pl.kernel,
        out_type=jax.ShapeDtypeStruct((NC, NPAD, 16), jnp.float32),
        mesh=mesh,
        compiler_params=pltpu.CompilerParams(use_tc_tiling_on_sc=False),
        scratch_types=[
            pltpu.VMEM((K, CHUNK), jnp.int32),
            pltpu.VMEM((K, CHUNK), jnp.int32),
            pltpu.VMEM((K, CHUNK, 16), jnp.float32),
            pltpu.SemaphoreType.DMA,
            pltpu.SemaphoreType.DMA,
            pltpu.VMEM_SHARED((NPAD, 16), jnp.float32),
        ],
    )
    def k(src_h, dst_h, tab_h, z_h, out_h,
          isv, idv, rows_v, gsem, ssem, acc):
        c = lax.axis_index("c")
        s = lax.axis_index("s")
        pltpu.sync_copy(z_h.at[pl.ds(s * RPT, RPT)], acc.at[pl.ds(s * RPT, RPT)])
        plsc.subcore_barrier()
        base = (c * NS + s) * CPT

        def body(g, carry):
            row0 = base + g * K
            pltpu.sync_copy(src_h.at[pl.ds(row0, K)], isv)
            pltpu.sync_copy(dst_h.at[pl.ds(row0, K)], idv)
            da = [pltpu.async_copy(tab_h.at[isv.at[j]], rows_v.at[j], gsem)
                  for j in range(K)]
            ds = []
            for j in range(K):
                da[j].wait()
                ds.append(pltpu.async_copy(rows_v.at[j], acc.at[idv.at[j]],
                                           ssem, add=True))
            for d in ds:
                d.wait()
            return carry

        lax.fori_loop(0, G, body, 0)
        plsc.subcore_barrier()
        pltpu.sync_copy(acc.at[pl.ds(s * RPT, RPT)],
                        out_h.at[c, pl.ds(s * RPT, RPT)])

    return k(src_r, dst_r, table, z16)


def _sc_agg32(src_r, dst_r, y1p, z32):
    K = 4
    G = CPS // K
    mesh = plsc.VectorSubcoreMesh(core_axis_name="c", subcore_axis_name="s")

    @functools.partial(
        pl.kernel,
        out_type=jax.ShapeDtypeStruct((NC, NPAD, 32), jnp.float32),
        mesh=mesh,
        compiler_params=pltpu.CompilerParams(use_tc_tiling_on_sc=False),
        scratch_types=[
            pltpu.VMEM((K, CHUNK), jnp.int32),
            pltpu.VMEM((K, CHUNK), jnp.int32),
            pltpu.VMEM((K, CHUNK, 32), jnp.float32),
            pltpu.SemaphoreType.DMA,
            pltpu.SemaphoreType.DMA,
            pltpu.VMEM_SHARED((NPAD, 32), jnp.float32),
        ],
    )
    def k(src_h, dst_h, tab_h, z_h, out_h,
          isv, idv, rows_v, gsem, ssem, acc):
        c = lax.axis_index("c")
        s = lax.axis_index("s")
        pltpu.sync_copy(z_h.at[pl.ds(s * RPT, RPT)], acc.at[pl.ds(s * RPT, RPT)])
        plsc.subcore_barrier()
        base = s * CPS
        tab_c = tab_h.at[c]

        def body(g, carry):
            row0 = base + g * K
            pltpu.sync_copy(src_h.at[pl.ds(row0, K)], isv)
            pltpu.sync_copy(dst_h.at[pl.ds(row0, K)], idv)
            da = [pltpu.async_copy(tab_c.at[isv.at[j]], rows_v.at[j], gsem)
                  for j in range(K)]
            ds = []
            for j in range(K):
                da[j].wait()
                ds.append(pltpu.async_copy(rows_v.at[j], acc.at[idv.at[j]],
                                           ssem, add=True))
            for d in ds:
                d.wait()
            return carry

        lax.fori_loop(0, G, body, 0)
        plsc.subcore_barrier()
        pltpu.sync_copy(acc.at[pl.ds(s * RPT, RPT)],
                        out_h.at[c, pl.ds(s * RPT, RPT)])

    return k(src_r, dst_r, y1p, z32)


def _sc_pool(h2pad, seg, z80):
    rpc = NPAD // NC
    rpt = rpc // NS
    nfull = rpt // CHUNK
    tail = rpt - nfull * CHUNK
    arows = (2 * B) // NS
    mesh = plsc.VectorSubcoreMesh(core_axis_name="c", subcore_axis_name="s")

    @functools.partial(
        pl.kernel,
        out_type=jax.ShapeDtypeStruct((NC, 2 * B, PW), jnp.float32),
        mesh=mesh,
        compiler_params=pltpu.CompilerParams(use_tc_tiling_on_sc=False),
        scratch_types=[
            pltpu.VMEM((CHUNK,), jnp.int32),
            pltpu.VMEM((tail,), jnp.int32),
            pltpu.VMEM((CHUNK, PW), jnp.float32),
            pltpu.VMEM((tail, PW), jnp.float32),
            pltpu.VMEM_SHARED((2 * B, PW), jnp.float32),
        ],
    )
    def k(h_h, seg_h, z_h, out_h, idx_v, idxt_v, rows_v, rowst_v, acc):
        c = lax.axis_index("c")
        s = lax.axis_index("s")
        pltpu.sync_copy(z_h.at[pl.ds(s * arows, arows)],
                        acc.at[pl.ds(s * arows, arows)])
        plsc.subcore_barrier()
        base = c * rpc + s * rpt

        def body(g, carry):
            off = base + g * CHUNK
            pltpu.sync_copy(seg_h.at[pl.ds(off, CHUNK)], idx_v)
            pltpu.sync_copy(h_h.at[pl.ds(off, CHUNK)], rows_v)
            pltpu.sync_copy(rows_v, acc.at[idx_v], add=True)
            return carry

        lax.fori_loop(0, nfull, body, 0)
        off = base + nfull * CHUNK
        pltpu.sync_copy(seg_h.at[pl.ds(off, tail)], idxt_v)
        pltpu.sync_copy(h_h.at[pl.ds(off, tail)], rowst_v)
        pltpu.sync_copy(rowst_v, acc.at[idxt_v], add=True)
        plsc.subcore_barrier()
        pltpu.sync_copy(acc.at[pl.ds(s * arows, arows)],
                        out_h.at[c, pl.ds(s * arows, arows)])

    return k(h2pad, seg, z80)


def _tc_prep(degp, x_pad, batch2d, gid2d):
    def body(degp_ref, x_ref, b_ref, g_ref, dinv_ref, y0_ref, seg_ref):
        deg16 = degp_ref[0] + degp_ref[1] + 1.0
        dinv16 = lax.rsqrt(deg16)
        dinv_ref[...] = dinv16
        y0 = x_ref[...] * dinv16[:, 0:2]
        y0_ref[...] = jnp.concatenate(
            [y0, jnp.zeros((BR, 14), jnp.float32)], axis=1)
        seg_ref[...] = b_ref[...] + B * g_ref[...]

    return pl.pallas_call(
        body,
        grid=(NB,),
        in_specs=[
            pl.BlockSpec((NC, BR, 16), lambda i: (0, i, 0)),
            pl.BlockSpec((BR, 2), lambda i: (i, 0)),
            pl.BlockSpec((1, 1, BR), lambda i: (i, 0, 0)),
            pl.BlockSpec((1, 1, BR), lambda i: (i, 0, 0)),
        ],
        out_specs=[
            pl.BlockSpec((BR, 16), lambda i: (i, 0)),
            pl.BlockSpec((BR, 16), lambda i: (i, 0)),
            pl.BlockSpec((1, 1, BR), lambda i: (i, 0, 0)),
        ],
        out_shape=[
            jax.ShapeDtypeStruct((NPAD, 16), jnp.float32),
            jax.ShapeDtypeStruct((NPAD, 16), jnp.float32),
            jax.ShapeDtypeStruct((NB, 1, BR), jnp.int32),
        ],
    )(degp, x_pad, batch2d, gid2d)


def _tc_layer1(s1p, y0pad, dinv16, W1, b1):
    def body(s1_ref, y0_ref, dv_ref, w1_ref, b1_ref, y1_ref):
        dinv16 = dv_ref[...]
        a1 = dinv16 * (s1_ref[0] + s1_ref[1] + y0_ref[...])
        h1 = a1[:, 0:1] * w1_ref[0:1, :] + a1[:, 1:2] * w1_ref[1:2, :]
        h1 = jnp.maximum(h1 + b1_ref[...], 0.0)
        y1 = dinv16[:, 0:1] * h1
        y1_ref[0] = y1[:, 0:32]
        y1_ref[1] = y1[:, 32:64]

    return pl.pallas_call(
        body,
        grid=(NB,),
        in_specs=[
            pl.BlockSpec((NC, BR, 16), lambda i: (0, i, 0)),
            pl.BlockSpec((BR, 16), lambda i: (i, 0)),
            pl.BlockSpec((BR, 16), lambda i: (i, 0)),
            pl.BlockSpec((2, H), lambda i: (0, 0)),
            pl.BlockSpec((1, H), lambda i: (0, 0)),
        ],
        out_specs=pl.BlockSpec((NC, BR, 32), lambda i: (0, i, 0)),
        out_shape=jax.ShapeDtypeStruct((NC, NPAD, 32), jnp.float32),
    )(s1p, y0pad, dinv16, W1, b1)


def _tc_layer2(s2p, y1p, dinv16, W2, b2):
    def body(s2_ref, y1_ref, dv_ref, w2_ref, b2_ref, out_ref):
        i = pl.program_id(0)
        s2 = jnp.concatenate([s2_ref[0] + y1_ref[0], s2_ref[1] + y1_ref[1]],
                             axis=1)
        a2 = dv_ref[...][:, 0:1] * s2
        h2 = jnp.dot(a2, w2_ref[...], preferred_element_type=jnp.float32)
        h2 = jnp.maximum(h2 + b2_ref[...], 0.0)
        row = i * BR + lax.broadcasted_iota(jnp.int32, (BR, 1), 0)
        valid = (row < N).astype(jnp.float32)
        out_ref[...] = jnp.concatenate(
            [h2 * valid, valid, jnp.zeros((BR, PW - H - 1), jnp.float32)],
            axis=1)

    return pl.pallas_call(
        body,
        grid=(NB,),
        in_specs=[
            pl.BlockSpec((NC, BR, 32), lambda i: (0, i, 0)),
            pl.BlockSpec((NC, BR, 32), lambda i: (0, i, 0)),
            pl.BlockSpec((BR, 16), lambda i: (i, 0)),
            pl.BlockSpec((H, H), lambda i: (0, 0)),
            pl.BlockSpec((1, H), lambda i: (0, 0)),
        ],
        out_specs=pl.BlockSpec((BR, PW), lambda i: (i, 0)),
        out_shape=jax.ShapeDtypeStruct((NPAD, PW), jnp.float32),
    )(s2p, y1p, dinv16, W2, b2)


def _tc_final(poolp, Wfc, bfc):
    def body(p_ref, w_ref, b_ref, out_ref):
        P = p_ref[0] + p_ref[1]
        sums = P[:, 0:H]
        cnt = P[:, H:H + 1]
        p = sums / jnp.maximum(cnt, 1.0)
        combined = jnp.concatenate([p[0:B], p[B:2 * B]], axis=1)
        out_ref[...] = jnp.dot(combined, w_ref[...],
                               preferred_element_type=jnp.float32) + b_ref[...]

    return pl.pallas_call(
        body,
        grid=(1,),
        in_specs=[
            pl.BlockSpec((NC, 2 * B, PW), lambda i: (0, 0, 0)),
            pl.BlockSpec((2 * H, NCLS), lambda i: (0, 0)),
            pl.BlockSpec((1, NCLS), lambda i: (0, 0)),
        ],
        out_specs=pl.BlockSpec((B, NCLS), lambda i: (0, 0)),
        out_shape=jax.ShapeDtypeStruct((B, NCLS), jnp.float32),
    )(poolp, Wfc, bfc)


def kernel(x, edge_index, graph_id, batch, W1, b1, W2, b2, Wfc, bfc):
    src = edge_index[0].astype(jnp.int32)
    dst = edge_index[1].astype(jnp.int32)
    src_r = jnp.concatenate(
        [src, jnp.zeros((EPAD - E,), jnp.int32)]).reshape(CR, CHUNK)
    dst_r = jnp.concatenate(
        [dst, jnp.full((EPAD - E,), N, jnp.int32)]).reshape(CR, CHUNK)
    x_pad = jnp.pad(x, ((0, NPAD - N), (0, 0)))
    batch2d = jnp.pad(batch, (0, NPAD - N)).reshape(NB, 1, BR)
    gid2d = jnp.pad(graph_id, (0, NPAD - N)).reshape(NB, 1, BR)
    ones_rows = jnp.ones((CHUNK, 16), jnp.float32)
    z16 = jnp.zeros((NPAD, 16), jnp.float32)
    z32 = jnp.zeros((NPAD, 32), jnp.float32)
    z80 = jnp.zeros((2 * B, PW), jnp.float32)

    degp = _sc_degree(dst_r, ones_rows, z16)
    dinv16, y0pad, seg2d = _tc_prep(degp, x_pad, batch2d, gid2d)
    s1p = _sc_agg16(src_r, dst_r, y0pad, z16)
    y1p = _tc_layer1(s1p, y0pad, dinv16, W1, b1.reshape(1, H))
    s2p = _sc_agg32(src_r, dst_r, y1p, z32)
    h2pad = _tc_layer2(s2p, y1p, dinv16, W2, b2.reshape(1, H))
    poolp = _sc_pool(h2pad, seg2d.reshape(NPAD), z80)
    return _tc_final(poolp, Wfc, bfc.reshape(1, NCLS))


# speedup vs baseline: 29.1620x; 1.0552x over previous
import functools

import jax
import jax.numpy as jnp
from jax import lax
from jax.experimental import pallas as pl
from jax.experimental.pallas import tpu as pltpu
from jax.experimental.pallas import tpu_sc as plsc

N = 50000
E = 800000
H = 64
B = 512
NCLS = 7
NC = 2
NS = 16
NPAD = 50176
RPT = NPAD // NS
NBLK = NPAD // 128
BR = 3136
NB = NPAD // BR
CHUNK = 128
EPAD = 802816
CR = EPAD // CHUNK
CPT = CR // (NC * NS)
CPS = CR // NS
PW = 80


def _sc_degree(dst_r, ones_rows, z16):
    K = 7
    G = CPT // K
    mesh = plsc.VectorSubcoreMesh(core_axis_name="c", subcore_axis_name="s")

    @functools.partial(
        pl.kernel,
        out_type=jax.ShapeDtypeStruct((NC, NPAD, 16), jnp.float32),
        mesh=mesh,
        compiler_params=pltpu.CompilerParams(use_tc_tiling_on_sc=False),
        scratch_types=[
            pltpu.VMEM((K, CHUNK), jnp.int32),
            pltpu.VMEM((CHUNK, 16), jnp.float32),
            pltpu.SemaphoreType.DMA,
            pltpu.VMEM_SHARED((NPAD, 16), jnp.float32),
        ],
    )
    def k(dst_h, ones_h, z_h, out_h, idx_v, rows_v, sem, acc):
        c = lax.axis_index("c")
        s = lax.axis_index("s")
        pltpu.sync_copy(z_h.at[pl.ds(s * RPT, RPT)], acc.at[pl.ds(s * RPT, RPT)])
        pltpu.sync_copy(ones_h, rows_v)
        plsc.subcore_barrier()
        base = (c * NS + s) * CPT

        def body(g, carry):
            row0 = base + g * K
            pltpu.sync_copy(dst_h.at[pl.ds(row0, K)], idx_v)
            ds = [pltpu.async_copy(rows_v, acc.at[idx_v.at[j]], sem, add=True)
                  for j in range(K)]
            for d in ds:
                d.wait()
            return carry

        lax.fori_loop(0, G, body, 0)
        plsc.subcore_barrier()
        pltpu.sync_copy(acc.at[pl.ds(s * RPT, RPT)],
                        out_h.at[c, pl.ds(s * RPT, RPT)])

    return k(dst_r, ones_rows, z16)


def _sc_agg16(src_r, dst_r, table, z16):
    K = 7
    G = CPT // K
    mesh = plsc.VectorSubcoreMesh(core_axis_name="c", subcore_axis_name="s")

    @functools.partial(
        pl.kernel,
        out_type=jax.ShapeDtypeStruct((NC, NPAD, 16), jnp.float32),
        mesh=mesh,
        compiler_params=pltpu.CompilerParams(use_tc_tiling_on_sc=False),
        scratch_types=[
            pltpu.VMEM((K, CHUNK), jnp.int32),
            pltpu.VMEM((K, CHUNK), jnp.int32),
            pltpu.VMEM((K, CHUNK, 16), jnp.float32),
            pltpu.SemaphoreType.DMA,
            pltpu.SemaphoreType.DMA,
            pltpu.VMEM_SHARED((NPAD, 16), jnp.float32),
        ],
    )
    def k(src_h, dst_h, tab_h, z_h, out_h,
          isv, idv, rows_v, gsem, ssem, acc):
        c = lax.axis_index("c")
        s = lax.axis_index("s")
        pltpu.sync_copy(z_h.at[pl.ds(s * RPT, RPT)], acc.at[pl.ds(s * RPT, RPT)])
        plsc.subcore_barrier()
        base = (c * NS + s) * CPT

        def body(g, carry):
            row0 = base + g * K
            pltpu.sync_copy(src_h.at[pl.ds(row0, K)], isv)
            pltpu.sync_copy(dst_h.at[pl.ds(row0, K)], idv)
            da = [pltpu.async_copy(tab_h.at[isv.at[j]], rows_v.at[j], gsem)
                  for j in range(K)]
            ds = []
            for j in range(K):
                da[j].wait()
                ds.append(pltpu.async_copy(rows_v.at[j], acc.at[idv.at[j]],
                                           ssem, add=True))
            for d in ds:
                d.wait()
            return carry

        lax.fori_loop(0, G, body, 0)
        plsc.subcore_barrier()
        pltpu.sync_copy(acc.at[pl.ds(s * RPT, RPT)],
                        out_h.at[c, pl.ds(s * RPT, RPT)])

    return k(src_r, dst_r, table, z16)


def _sc_agg32(src_r, dst_r, y1p, z32):
    K = 4
    G = CPS // K
    mesh = plsc.VectorSubcoreMesh(core_axis_name="c", subcore_axis_name="s")

    @functools.partial(
        pl.kernel,
        out_type=jax.ShapeDtypeStruct((NC, NPAD, 32), jnp.float32),
        mesh=mesh,
        compiler_params=pltpu.CompilerParams(use_tc_tiling_on_sc=False),
        scratch_types=[
            pltpu.VMEM((K, CHUNK), jnp.int32),
            pltpu.VMEM((K, CHUNK), jnp.int32),
            pltpu.VMEM((K, CHUNK, 32), jnp.float32),
            pltpu.SemaphoreType.DMA,
            pltpu.SemaphoreType.DMA,
            pltpu.VMEM_SHARED((NPAD, 32), jnp.float32),
        ],
    )
    def k(src_h, dst_h, tab_h, z_h, out_h,
          isv, idv, rows_v, gsem, ssem, acc):
        c = lax.axis_index("c")
        s = lax.axis_index("s")
        pltpu.sync_copy(z_h.at[pl.ds(s * RPT, RPT)], acc.at[pl.ds(s * RPT, RPT)])
        plsc.subcore_barrier()
        base = s * CPS
        tab_c = tab_h.at[c]

        def body(g, carry):
            row0 = base + g * K
            pltpu.sync_copy(src_h.at[pl.ds(row0, K)], isv)
            pltpu.sync_copy(dst_h.at[pl.ds(row0, K)], idv)
            da = [pltpu.async_copy(tab_c.at[isv.at[j]], rows_v.at[j], gsem)
                  for j in range(K)]
            ds = []
            for j in range(K):
                da[j].wait()
                ds.append(pltpu.async_copy(rows_v.at[j], acc.at[idv.at[j]],
                                           ssem, add=True))
            for d in ds:
                d.wait()
            return carry

        lax.fori_loop(0, G, body, 0)
        plsc.subcore_barrier()
        pltpu.sync_copy(acc.at[pl.ds(s * RPT, RPT)],
                        out_h.at[c, pl.ds(s * RPT, RPT)])

    return k(src_r, dst_r, y1p, z32)


def _sc_pool(h2pad, seg, z80):
    rpc = NPAD // NC
    rpt = rpc // NS
    nfull = rpt // CHUNK
    tail = rpt - nfull * CHUNK
    arows = (2 * B) // NS
    mesh = plsc.VectorSubcoreMesh(core_axis_name="c", subcore_axis_name="s")

    @functools.partial(
        pl.kernel,
        out_type=jax.ShapeDtypeStruct((NC, 2 * B, PW), jnp.float32),
        mesh=mesh,
        compiler_params=pltpu.CompilerParams(use_tc_tiling_on_sc=False),
        scratch_types=[
            pltpu.VMEM((CHUNK,), jnp.int32),
            pltpu.VMEM((tail,), jnp.int32),
            pltpu.VMEM((CHUNK, PW), jnp.float32),
            pltpu.VMEM((tail, PW), jnp.float32),
            pltpu.VMEM_SHARED((2 * B, PW), jnp.float32),
        ],
    )
    def k(h_h, seg_h, z_h, out_h, idx_v, idxt_v, rows_v, rowst_v, acc):
        c = lax.axis_index("c")
        s = lax.axis_index("s")
        pltpu.sync_copy(z_h.at[pl.ds(s * arows, arows)],
                        acc.at[pl.ds(s * arows, arows)])
        plsc.subcore_barrier()
        base = c * rpc + s * rpt

        def body(g, carry):
            off = base + g * CHUNK
            pltpu.sync_copy(seg_h.at[pl.ds(off, CHUNK)], idx_v)
            pltpu.sync_copy(h_h.at[pl.ds(off, CHUNK)], rows_v)
            pltpu.sync_copy(rows_v, acc.at[idx_v], add=True)
            return carry

        lax.fori_loop(0, nfull, body, 0)
        off = base + nfull * CHUNK
        pltpu.sync_copy(seg_h.at[pl.ds(off, tail)], idxt_v)
        pltpu.sync_copy(h_h.at[pl.ds(off, tail)], rowst_v)
        pltpu.sync_copy(rowst_v, acc.at[idxt_v], add=True)
        plsc.subcore_barrier()
        pltpu.sync_copy(acc.at[pl.ds(s * arows, arows)],
                        out_h.at[c, pl.ds(s * arows, arows)])

    return k(h2pad, seg, z80)


def _tc_prep(degp, x_pad, batch2d, gid2d):
    def body(degp_ref, x_ref, b_ref, g_ref, dinv_ref, y0_ref, seg_ref):
        deg16 = degp_ref[0] + degp_ref[1] + 1.0
        dinv16 = lax.rsqrt(deg16)
        dinv_ref[...] = dinv16
        y0 = x_ref[...] * dinv16[:, 0:2]
        y0_ref[...] = jnp.concatenate(
            [y0, jnp.zeros((BR, 14), jnp.float32)], axis=1)
        seg_ref[...] = b_ref[...] + B * g_ref[...]

    return pl.pallas_call(
        body,
        grid=(NB,),
        in_specs=[
            pl.BlockSpec((NC, BR, 16), lambda i: (0, i, 0)),
            pl.BlockSpec((BR, 2), lambda i: (i, 0)),
            pl.BlockSpec((1, 1, BR), lambda i: (i, 0, 0)),
            pl.BlockSpec((1, 1, BR), lambda i: (i, 0, 0)),
        ],
        out_specs=[
            pl.BlockSpec((BR, 16), lambda i: (i, 0)),
            pl.BlockSpec((BR, 16), lambda i: (i, 0)),
            pl.BlockSpec((1, 1, BR), lambda i: (i, 0, 0)),
        ],
        out_shape=[
            jax.ShapeDtypeStruct((NPAD, 16), jnp.float32),
            jax.ShapeDtypeStruct((NPAD, 16), jnp.float32),
            jax.ShapeDtypeStruct((NB, 1, BR), jnp.int32),
        ],
    )(degp, x_pad, batch2d, gid2d)


def _tc_layer1(s1p, y0pad, dinv16, W1, b1):
    def body(s1_ref, y0_ref, dv_ref, w1_ref, b1_ref, y1_ref):
        dinv16 = dv_ref[...]
        a1 = dinv16 * (s1_ref[0] + s1_ref[1] + y0_ref[...])
        h1 = a1[:, 0:1] * w1_ref[0:1, :] + a1[:, 1:2] * w1_ref[1:2, :]
        h1 = jnp.maximum(h1 + b1_ref[...], 0.0)
        y1 = dinv16[:, 0:1] * h1
        y1_ref[0] = y1[:, 0:32]
        y1_ref[1] = y1[:, 32:64]

    return pl.pallas_call(
        body,
        grid=(NB,),
        in_specs=[
            pl.BlockSpec((NC, BR, 16), lambda i: (0, i, 0)),
            pl.BlockSpec((BR, 16), lambda i: (i, 0)),
            pl.BlockSpec((BR, 16), lambda i: (i, 0)),
            pl.BlockSpec((2, H), lambda i: (0, 0)),
            pl.BlockSpec((1, H), lambda i: (0, 0)),
        ],
        out_specs=pl.BlockSpec((NC, BR, 32), lambda i: (0, i, 0)),
        out_shape=jax.ShapeDtypeStruct((NC, NPAD, 32), jnp.float32),
    )(s1p, y0pad, dinv16, W1, b1)


def _tc_layer2(s2p, y1p, dinv16, W2, b2):
    def body(s2_ref, y1_ref, dv_ref, w2_ref, b2_ref, out_ref):
        i = pl.program_id(0)
        s2 = jnp.concatenate([s2_ref[0] + y1_ref[0], s2_ref[1] + y1_ref[1]],
                             axis=1)
        a2 = dv_ref[...][:, 0:1] * s2
        h2 = jnp.dot(a2, w2_ref[...], preferred_element_type=jnp.float32)
        h2 = jnp.maximum(h2 + b2_ref[...], 0.0)
        row = i * BR + lax.broadcasted_iota(jnp.int32, (BR, 1), 0)
        valid = (row < N).astype(jnp.float32)
        out_ref[...] = jnp.concatenate(
            [h2 * valid, valid, jnp.zeros((BR, PW - H - 1), jnp.float32)],
            axis=1)

    return pl.pallas_call(
        body,
        grid=(NB,),
        in_specs=[
            pl.BlockSpec((NC, BR, 32), lambda i: (0, i, 0)),
            pl.BlockSpec((NC, BR, 32), lambda i: (0, i, 0)),
            pl.BlockSpec((BR, 16), lambda i: (i, 0)),
            pl.BlockSpec((H, H), lambda i: (0, 0)),
            pl.BlockSpec((1, H), lambda i: (0, 0)),
        ],
        out_specs=pl.BlockSpec((BR, PW), lambda i: (i, 0)),
        out_shape=jax.ShapeDtypeStruct((NPAD, PW), jnp.float32),
    )(s2p, y1p, dinv16, W2, b2)


def _tc_final(poolp, Wfc, bfc):
    def body(p_ref, w_ref, b_ref, out_ref):
        P = p_ref[0] + p_ref[1]
        sums = P[:, 0:H]
        cnt = P[:, H:H + 1]
        p = sums / jnp.maximum(cnt, 1.0)
        combined = jnp.concatenate([p[0:B], p[B:2 * B]], axis=1)
        out_ref[...] = jnp.dot(combined, w_ref[...],
                               preferred_element_type=jnp.float32) + b_ref[...]

    return pl.pallas_call(
        body,
        grid=(1,),
        in_specs=[
            pl.BlockSpec((NC, 2 * B, PW), lambda i: (0, 0, 0)),
            pl.BlockSpec((2 * H, NCLS), lambda i: (0, 0)),
            pl.BlockSpec((1, NCLS), lambda i: (0, 0)),
        ],
        out_specs=pl.BlockSpec((B, NCLS), lambda i: (0, 0)),
        out_shape=jax.ShapeDtypeStruct((B, NCLS), jnp.float32),
    )(poolp, Wfc, bfc)


def kernel(x, edge_index, graph_id, batch, W1, b1, W2, b2, Wfc, bfc):
    src = edge_index[0].astype(jnp.int32)
    dst = edge_index[1].astype(jnp.int32)
    src_r = jnp.concatenate(
        [src, jnp.zeros((EPAD - E,), jnp.int32)]).reshape(CR, CHUNK)
    dst_r = jnp.concatenate(
        [dst, jnp.full((EPAD - E,), N, jnp.int32)]).reshape(CR, CHUNK)
    x_pad = jnp.pad(x, ((0, NPAD - N), (0, 0)))
    batch2d = jnp.pad(batch, (0, NPAD - N)).reshape(NB, 1, BR)
    gid2d = jnp.pad(graph_id, (0, NPAD - N)).reshape(NB, 1, BR)
    ones_rows = jnp.ones((CHUNK, 16), jnp.float32)
    z16 = jnp.zeros((NPAD, 16), jnp.float32)
    z32 = jnp.zeros((NPAD, 32), jnp.float32)
    z80 = jnp.zeros((2 * B, PW), jnp.float32)

    degp = _sc_degree(dst_r, ones_rows, z16)
    dinv16, y0pad, seg2d = _tc_prep(degp, x_pad, batch2d, gid2d)
    s1p = _sc_agg16(src_r, dst_r, y0pad, z16)
    y1p = _tc_layer1(s1p, y0pad, dinv16, W1, b1.reshape(1, H))
    s2p = _sc_agg32(src_r, dst_r, y1p, z32)
    h2pad = _tc_layer2(s2p, y1p, dinv16, W2, b2.reshape(1, H))
    poolp = _sc_pool(h2pad, seg2d.reshape(NPAD), z80)
    return _tc_final(poolp, Wfc, bfc.reshape(1, NCLS))


# speedup vs baseline: 31.2354x; 1.0711x over previous
import functools

import jax
import jax.numpy as jnp
from jax import lax
from jax.experimental import pallas as pl
from jax.experimental.pallas import tpu as pltpu
from jax.experimental.pallas import tpu_sc as plsc

N = 50000
E = 800000
H = 64
B = 512
NCLS = 7
NC = 2
NS = 16
NPAD = 50176
RPT = NPAD // NS
NBLK = NPAD // 128
BR = 3136
NB = NPAD // BR
CHUNK = 128
EPAD = 802816
CR = EPAD // CHUNK
CPT = CR // (NC * NS)
CPS = CR // NS
PW = 80


def _sc_degree(dst_r, ones_rows, z16):
    K = 7
    G = CPT // K
    mesh = plsc.VectorSubcoreMesh(core_axis_name="c", subcore_axis_name="s")

    @functools.partial(
        pl.kernel,
        out_type=jax.ShapeDtypeStruct((NC, NPAD, 16), jnp.float32),
        mesh=mesh,
        compiler_params=pltpu.CompilerParams(use_tc_tiling_on_sc=False),
        scratch_types=[
            pltpu.VMEM((K, CHUNK), jnp.int32),
            pltpu.VMEM((CHUNK, 16), jnp.float32),
            pltpu.SemaphoreType.DMA,
            pltpu.VMEM_SHARED((NPAD, 16), jnp.float32),
        ],
    )
    def k(dst_h, ones_h, z_h, out_h, idx_v, rows_v, sem, acc):
        c = lax.axis_index("c")
        s = lax.axis_index("s")
        pltpu.sync_copy(z_h.at[pl.ds(s * RPT, RPT)], acc.at[pl.ds(s * RPT, RPT)])
        pltpu.sync_copy(ones_h, rows_v)
        plsc.subcore_barrier()
        base = (c * NS + s) * CPT

        def body(g, carry):
            row0 = base + g * K
            pltpu.sync_copy(dst_h.at[pl.ds(row0, K)], idx_v)
            ds = [pltpu.async_copy(rows_v, acc.at[idx_v.at[j]], sem, add=True)
                  for j in range(K)]
            for d in ds:
                d.wait()
            return carry

        lax.fori_loop(0, G, body, 0)
        plsc.subcore_barrier()
        pltpu.sync_copy(acc.at[pl.ds(s * RPT, RPT)],
                        out_h.at[c, pl.ds(s * RPT, RPT)])

    return k(dst_r, ones_rows, z16)


def _sc_agg16(sd_r, table, z16):
    K = 7
    G = CPT // K
    mesh = plsc.VectorSubcoreMesh(core_axis_name="c", subcore_axis_name="s")

    @functools.partial(
        pl.kernel,
        out_type=jax.ShapeDtypeStruct((NC, NPAD, 16), jnp.float32),
        mesh=mesh,
        compiler_params=pltpu.CompilerParams(use_tc_tiling_on_sc=False),
        scratch_types=[
            pltpu.VMEM((K, 2, CHUNK), jnp.int32),
            pltpu.VMEM((K, CHUNK, 16), jnp.float32),
            pltpu.SemaphoreType.DMA,
            pltpu.SemaphoreType.DMA,
            pltpu.VMEM_SHARED((NPAD, 16), jnp.float32),
        ],
    )
    def k(sd_h, tab_h, z_h, out_h, iv, rows_v, gsem, ssem, acc):
        c = lax.axis_index("c")
        s = lax.axis_index("s")
        pltpu.sync_copy(z_h.at[pl.ds(s * RPT, RPT)], acc.at[pl.ds(s * RPT, RPT)])
        plsc.subcore_barrier()
        base = (c * NS + s) * CPT

        def body(g, carry):
            row0 = base + g * K
            pltpu.sync_copy(sd_h.at[pl.ds(row0, K)], iv)
            da = [pltpu.async_copy(tab_h.at[iv.at[j, 0]], rows_v.at[j], gsem)
                  for j in range(K)]
            ds = []
            for j in range(K):
                da[j].wait()
                ds.append(pltpu.async_copy(rows_v.at[j], acc.at[iv.at[j, 1]],
                                           ssem, add=True))
            for d in ds:
                d.wait()
            return carry

        lax.fori_loop(0, G, body, 0)
        plsc.subcore_barrier()
        pltpu.sync_copy(acc.at[pl.ds(s * RPT, RPT)],
                        out_h.at[c, pl.ds(s * RPT, RPT)])

    return k(sd_r, table, z16)


def _sc_agg32(sd_r, y1p, z32):
    K = 4
    G = CPS // K
    mesh = plsc.VectorSubcoreMesh(core_axis_name="c", subcore_axis_name="s")

    @functools.partial(
        pl.kernel,
        out_type=jax.ShapeDtypeStruct((NC, NPAD, 32), jnp.float32),
        mesh=mesh,
        compiler_params=pltpu.CompilerParams(use_tc_tiling_on_sc=False),
        scratch_types=[
            pltpu.VMEM((K, 2, CHUNK), jnp.int32),
            pltpu.VMEM((K, CHUNK, 32), jnp.float32),
            pltpu.SemaphoreType.DMA,
            pltpu.SemaphoreType.DMA,
            pltpu.VMEM_SHARED((NPAD, 32), jnp.float32),
        ],
    )
    def k(sd_h, tab_h, z_h, out_h, iv, rows_v, gsem, ssem, acc):
        c = lax.axis_index("c")
        s = lax.axis_index("s")
        pltpu.sync_copy(z_h.at[pl.ds(s * RPT, RPT)], acc.at[pl.ds(s * RPT, RPT)])
        plsc.subcore_barrier()
        base = s * CPS
        tab_c = tab_h.at[c]

        def body(g, carry):
            row0 = base + g * K
            pltpu.sync_copy(sd_h.at[pl.ds(row0, K)], iv)
            da = [pltpu.async_copy(tab_c.at[iv.at[j, 0]], rows_v.at[j], gsem)
                  for j in range(K)]
            ds = []
            for j in range(K):
                da[j].wait()
                ds.append(pltpu.async_copy(rows_v.at[j], acc.at[iv.at[j, 1]],
                                           ssem, add=True))
            for d in ds:
                d.wait()
            return carry

        lax.fori_loop(0, G, body, 0)
        plsc.subcore_barrier()
        pltpu.sync_copy(acc.at[pl.ds(s * RPT, RPT)],
                        out_h.at[c, pl.ds(s * RPT, RPT)])

    return k(sd_r, y1p, z32)


def _sc_pool(h2pad, seg, z80):
    rpc = NPAD // NC
    rpt = rpc // NS
    nfull = rpt // CHUNK
    tail = rpt - nfull * CHUNK
    arows = (2 * B) // NS
    mesh = plsc.VectorSubcoreMesh(core_axis_name="c", subcore_axis_name="s")

    @functools.partial(
        pl.kernel,
        out_type=jax.ShapeDtypeStruct((NC, 2 * B, PW), jnp.float32),
        mesh=mesh,
        compiler_params=pltpu.CompilerParams(use_tc_tiling_on_sc=False),
        scratch_types=[
            pltpu.VMEM((CHUNK,), jnp.int32),
            pltpu.VMEM((tail,), jnp.int32),
            pltpu.VMEM((CHUNK, PW), jnp.float32),
            pltpu.VMEM((tail, PW), jnp.float32),
            pltpu.VMEM_SHARED((2 * B, PW), jnp.float32),
        ],
    )
    def k(h_h, seg_h, z_h, out_h, idx_v, idxt_v, rows_v, rowst_v, acc):
        c = lax.axis_index("c")
        s = lax.axis_index("s")
        pltpu.sync_copy(z_h.at[pl.ds(s * arows, arows)],
                        acc.at[pl.ds(s * arows, arows)])
        plsc.subcore_barrier()
        base = c * rpc + s * rpt

        def body(g, carry):
            off = base + g * CHUNK
            pltpu.sync_copy(seg_h.at[pl.ds(off, CHUNK)], idx_v)
            pltpu.sync_copy(h_h.at[pl.ds(off, CHUNK)], rows_v)
            pltpu.sync_copy(rows_v, acc.at[idx_v], add=True)
            return carry

        lax.fori_loop(0, nfull, body, 0)
        off = base + nfull * CHUNK
        pltpu.sync_copy(seg_h.at[pl.ds(off, tail)], idxt_v)
        pltpu.sync_copy(h_h.at[pl.ds(off, tail)], rowst_v)
        pltpu.sync_copy(rowst_v, acc.at[idxt_v], add=True)
        plsc.subcore_barrier()
        pltpu.sync_copy(acc.at[pl.ds(s * arows, arows)],
                        out_h.at[c, pl.ds(s * arows, arows)])

    return k(h2pad, seg, z80)


def _tc_prep(degp, x_pad, batch2d, gid2d):
    def body(degp_ref, x_ref, b_ref, g_ref, dinv_ref, y0_ref, seg_ref):
        deg16 = degp_ref[0] + degp_ref[1] + 1.0
        dinv16 = lax.rsqrt(deg16)
        dinv_ref[...] = dinv16
        y0 = x_ref[...] * dinv16[:, 0:2]
        y0_ref[...] = jnp.concatenate(
            [y0, jnp.zeros((BR, 14), jnp.float32)], axis=1)
        seg_ref[...] = b_ref[...] + B * g_ref[...]

    return pl.pallas_call(
        body,
        grid=(NB,),
        in_specs=[
            pl.BlockSpec((NC, BR, 16), lambda i: (0, i, 0)),
            pl.BlockSpec((BR, 2), lambda i: (i, 0)),
            pl.BlockSpec((1, 1, BR), lambda i: (i, 0, 0)),
            pl.BlockSpec((1, 1, BR), lambda i: (i, 0, 0)),
        ],
        out_specs=[
            pl.BlockSpec((BR, 16), lambda i: (i, 0)),
            pl.BlockSpec((BR, 16), lambda i: (i, 0)),
            pl.BlockSpec((1, 1, BR), lambda i: (i, 0, 0)),
        ],
        out_shape=[
            jax.ShapeDtypeStruct((NPAD, 16), jnp.float32),
            jax.ShapeDtypeStruct((NPAD, 16), jnp.float32),
            jax.ShapeDtypeStruct((NB, 1, BR), jnp.int32),
        ],
    )(degp, x_pad, batch2d, gid2d)


def _tc_layer1(s1p, y0pad, dinv16, W1, b1):
    def body(s1_ref, y0_ref, dv_ref, w1_ref, b1_ref, y1_ref):
        dinv16 = dv_ref[...]
        a1 = dinv16 * (s1_ref[0] + s1_ref[1] + y0_ref[...])
        h1 = a1[:, 0:1] * w1_ref[0:1, :] + a1[:, 1:2] * w1_ref[1:2, :]
        h1 = jnp.maximum(h1 + b1_ref[...], 0.0)
        y1 = dinv16[:, 0:1] * h1
        y1_ref[0] = y1[:, 0:32]
        y1_ref[1] = y1[:, 32:64]

    return pl.pallas_call(
        body,
        grid=(NB,),
        in_specs=[
            pl.BlockSpec((NC, BR, 16), lambda i: (0, i, 0)),
            pl.BlockSpec((BR, 16), lambda i: (i, 0)),
            pl.BlockSpec((BR, 16), lambda i: (i, 0)),
            pl.BlockSpec((2, H), lambda i: (0, 0)),
            pl.BlockSpec((1, H), lambda i: (0, 0)),
        ],
        out_specs=pl.BlockSpec((NC, BR, 32), lambda i: (0, i, 0)),
        out_shape=jax.ShapeDtypeStruct((NC, NPAD, 32), jnp.float32),
    )(s1p, y0pad, dinv16, W1, b1)


def _tc_layer2(s2p, y1p, dinv16, W2, b2):
    def body(s2_ref, y1_ref, dv_ref, w2_ref, b2_ref, out_ref):
        i = pl.program_id(0)
        s2 = jnp.concatenate([s2_ref[0] + y1_ref[0], s2_ref[1] + y1_ref[1]],
                             axis=1)
        a2 = dv_ref[...][:, 0:1] * s2
        h2 = jnp.dot(a2, w2_ref[...], preferred_element_type=jnp.float32)
        h2 = jnp.maximum(h2 + b2_ref[...], 0.0)
        row = i * BR + lax.broadcasted_iota(jnp.int32, (BR, 1), 0)
        valid = (row < N).astype(jnp.float32)
        out_ref[...] = jnp.concatenate(
            [h2 * valid, valid, jnp.zeros((BR, PW - H - 1), jnp.float32)],
            axis=1)

    return pl.pallas_call(
        body,
        grid=(NB,),
        in_specs=[
            pl.BlockSpec((NC, BR, 32), lambda i: (0, i, 0)),
            pl.BlockSpec((NC, BR, 32), lambda i: (0, i, 0)),
            pl.BlockSpec((BR, 16), lambda i: (i, 0)),
            pl.BlockSpec((H, H), lambda i: (0, 0)),
            pl.BlockSpec((1, H), lambda i: (0, 0)),
        ],
        out_specs=pl.BlockSpec((BR, PW), lambda i: (i, 0)),
        out_shape=jax.ShapeDtypeStruct((NPAD, PW), jnp.float32),
    )(s2p, y1p, dinv16, W2, b2)


def _tc_final(poolp, Wfc, bfc):
    def body(p_ref, w_ref, b_ref, out_ref):
        P = p_ref[0] + p_ref[1]
        sums = P[:, 0:H]
        cnt = P[:, H:H + 1]
        p = sums / jnp.maximum(cnt, 1.0)
        combined = jnp.concatenate([p[0:B], p[B:2 * B]], axis=1)
        out_ref[...] = jnp.dot(combined, w_ref[...],
                               preferred_element_type=jnp.float32) + b_ref[...]

    return pl.pallas_call(
        body,
        grid=(1,),
        in_specs=[
            pl.BlockSpec((NC, 2 * B, PW), lambda i: (0, 0, 0)),
            pl.BlockSpec((2 * H, NCLS), lambda i: (0, 0)),
            pl.BlockSpec((1, NCLS), lambda i: (0, 0)),
        ],
        out_specs=pl.BlockSpec((B, NCLS), lambda i: (0, 0)),
        out_shape=jax.ShapeDtypeStruct((B, NCLS), jnp.float32),
    )(poolp, Wfc, bfc)


def kernel(x, edge_index, graph_id, batch, W1, b1, W2, b2, Wfc, bfc):
    src = edge_index[0].astype(jnp.int32)
    dst = edge_index[1].astype(jnp.int32)
    src_r = jnp.concatenate(
        [src, jnp.zeros((EPAD - E,), jnp.int32)]).reshape(CR, CHUNK)
    dst_r = jnp.concatenate(
        [dst, jnp.full((EPAD - E,), N, jnp.int32)]).reshape(CR, CHUNK)
    x_pad = jnp.pad(x, ((0, NPAD - N), (0, 0)))
    batch2d = jnp.pad(batch, (0, NPAD - N)).reshape(NB, 1, BR)
    gid2d = jnp.pad(graph_id, (0, NPAD - N)).reshape(NB, 1, BR)
    ones_rows = jnp.ones((CHUNK, 16), jnp.float32)
    z16 = jnp.zeros((NPAD, 16), jnp.float32)
    z32 = jnp.zeros((NPAD, 32), jnp.float32)
    z80 = jnp.zeros((2 * B, PW), jnp.float32)

    sd_r = jnp.stack([src_r, dst_r], axis=1)
    degp = _sc_degree(dst_r, ones_rows, z16)
    dinv16, y0pad, seg2d = _tc_prep(degp, x_pad, batch2d, gid2d)
    s1p = _sc_agg16(sd_r, y0pad, z16)
    y1p = _tc_layer1(s1p, y0pad, dinv16, W1, b1.reshape(1, H))
    s2p = _sc_agg32(sd_r, y1p, z32)
    h2pad = _tc_layer2(s2p, y1p, dinv16, W2, b2.reshape(1, H))
    poolp = _sc_pool(h2pad, seg2d.reshape(NPAD), z80)
    return _tc_final(poolp, Wfc, bfc.reshape(1, NCLS))
